# Initial kernel scaffold; baseline (speedup 1.0000x reference)
import functools

import jax
import jax.numpy as jnp
from jax import lax
from jax.experimental import pallas as pl
from jax.experimental.pallas import tpu as pltpu
from jax.experimental.pallas import tpu_sc as plsc

N = 10000
E = 320000
EMB = 128
NLAYER = 5
NGRAPH = 64
NCLASS = 128

NC = 2
NS = 16
NW = NC * NS
EW = E // NW
WIN = 80
NWIN = EW // WIN
RPT = N // NS

BS = 1000
NBLK = N // BS

_mesh = plsc.VectorSubcoreMesh(core_axis_name="c", subcore_axis_name="s")


@functools.partial(
    pl.kernel,
    out_type=jax.ShapeDtypeStruct((NC * N, 16), jnp.float32),
    mesh=_mesh,
    scratch_types=[
        pltpu.VMEM((WIN,), jnp.int32),
        pltpu.VMEM((WIN, 16), jnp.float32),
        pltpu.VMEM_SHARED((N, 16), jnp.float32),
    ],
)
def _sc_degree(src_hbm, ones_hbm, zeros_hbm, out_hbm, sidx, ones_v, acc):
    c = lax.axis_index("c")
    s = lax.axis_index("s")
    pltpu.sync_copy(zeros_hbm.at[pl.ds(s * RPT, RPT)], acc.at[pl.ds(s * RPT, RPT)])
    pltpu.sync_copy(ones_hbm, ones_v)
    plsc.subcore_barrier()
    wid = c * NS + s

    @pl.loop(0, NWIN)
    def _(w):
        base = wid * EW + w * WIN
        pltpu.sync_copy(src_hbm.at[pl.ds(base, WIN)], sidx)
        pltpu.sync_copy(ones_v, acc.at[sidx], add=True)

    plsc.subcore_barrier()
    pltpu.sync_copy(acc.at[pl.ds(s * RPT, RPT)],
                    out_hbm.at[pl.ds(c * N + s * RPT, RPT)])


@functools.partial(
    pl.kernel,
    out_type=jax.ShapeDtypeStruct((NC * N, EMB), jnp.float32),
    mesh=_mesh,
    scratch_types=[
        pltpu.VMEM((WIN,), jnp.int32),
        pltpu.VMEM((WIN,), jnp.int32),
        pltpu.VMEM((WIN, EMB), jnp.float32),
        pltpu.VMEM_SHARED((N, EMB), jnp.float32),
        pltpu.SemaphoreType.DMA,
    ],
)
def _sc_aggregate(r2_hbm, src_hbm, dst_hbm, zeros_hbm, out_hbm,
                  sidx, didx, rows, acc, sem):
    c = lax.axis_index("c")
    s = lax.axis_index("s")
    pltpu.sync_copy(zeros_hbm.at[pl.ds(s * RPT, RPT)], acc.at[pl.ds(s * RPT, RPT)])
    plsc.subcore_barrier()
    wid = c * NS + s

    @pl.loop(0, NWIN)
    def _(w):
        base = wid * EW + w * WIN
        pltpu.sync_copy(src_hbm.at[pl.ds(base, WIN)], sidx)
        pltpu.sync_copy(dst_hbm.at[pl.ds(base, WIN)], didx)
        pltpu.async_copy(r2_hbm.at[sidx], rows, sem).wait()
        pltpu.sync_copy(rows, acc.at[didx], add=True)

    plsc.subcore_barrier()
    pltpu.sync_copy(acc.at[pl.ds(s * RPT, RPT)],
                    out_hbm.at[pl.ds(c * N + s * RPT, RPT)])


def _deg_body(p0_ref, p1_ref, dis_ref, inv_ref):
    deg = p0_ref[:, 0:1] + p1_ref[:, 0:1] + 1.0
    dis_ref[...] = lax.rsqrt(deg)
    inv_ref[...] = 1.0 / deg


def _tc_deg(hist):
    return pl.pallas_call(
        _deg_body,
        grid=(NBLK,),
        in_specs=[
            pl.BlockSpec((BS, 16), lambda i: (i, 0)),
            pl.BlockSpec((BS, 16), lambda i: (NBLK + i, 0)),
        ],
        out_specs=[
            pl.BlockSpec((BS, 1), lambda i: (i, 0)),
            pl.BlockSpec((BS, 1), lambda i: (i, 0)),
        ],
        out_shape=[
            jax.ShapeDtypeStruct((N, 1), jnp.float32),
            jax.ShapeDtypeStruct((N, 1), jnp.float32),
        ],
    )(hist, hist)


def _first_body(x_ref, w_ref, b_ref, root_ref, dis_ref, inv_ref,
                r2_ref, self_ref):
    hl = jnp.dot(x_ref[...], w_ref[...],
                 preferred_element_type=jnp.float32) + b_ref[...]
    r2_ref[...] = dis_ref[...] * jnp.maximum(hl, 0.0)
    self_ref[...] = jnp.maximum(hl + root_ref[...], 0.0) * inv_ref[...]


def _tc_first(x, Wl, bl, rootl, dis, invdeg):
    return pl.pallas_call(
        _first_body,
        grid=(NBLK,),
        in_specs=[
            pl.BlockSpec((BS, EMB), lambda i: (i, 0)),
            pl.BlockSpec((EMB, EMB), lambda i: (0, 0)),
            pl.BlockSpec((1, EMB), lambda i: (0, 0)),
            pl.BlockSpec((1, EMB), lambda i: (0, 0)),
            pl.BlockSpec((BS, 1), lambda i: (i, 0)),
            pl.BlockSpec((BS, 1), lambda i: (i, 0)),
        ],
        out_specs=[
            pl.BlockSpec((BS, EMB), lambda i: (i, 0)),
            pl.BlockSpec((BS, EMB), lambda i: (i, 0)),
        ],
        out_shape=[
            jax.ShapeDtypeStruct((N, EMB), jnp.float32),
            jax.ShapeDtypeStruct((N, EMB), jnp.float32),
        ],
    )(x, Wl, bl, rootl, dis, invdeg)


def _bn_apply(t, stats_ref, scale_ref, bias_ref):
    mean = stats_ref[0:1, :] * (1.0 / N)
    ex2 = stats_ref[1:2, :] * (1.0 / N)
    var = ex2 - mean * mean
    rstd = lax.rsqrt(var + 1e-5)
    return (t - mean) * rstd * scale_ref[...] + bias_ref[...]


def _mid_body(t_ref, stats_ref, scale_ref, bias_ref, w_ref, b_ref, root_ref,
              dis_ref, inv_ref, r2_ref, self_ref):
    h = jnp.maximum(_bn_apply(t_ref[...], stats_ref, scale_ref, bias_ref), 0.0)
    hl = jnp.dot(h, w_ref[...], preferred_element_type=jnp.float32) + b_ref[...]
    r2_ref[...] = dis_ref[...] * jnp.maximum(hl, 0.0)
    self_ref[...] = jnp.maximum(hl + root_ref[...], 0.0) * inv_ref[...]


def _tc_mid(t, stats, scale_prev, bias_prev, Wl, bl, rootl, dis, invdeg):
    return pl.pallas_call(
        _mid_body,
        grid=(NBLK,),
        in_specs=[
            pl.BlockSpec((BS, EMB), lambda i: (i, 0)),
            pl.BlockSpec((2, EMB), lambda i: (0, 0)),
            pl.BlockSpec((1, EMB), lambda i: (0, 0)),
            pl.BlockSpec((1, EMB), lambda i: (0, 0)),
            pl.BlockSpec((EMB, EMB), lambda i: (0, 0)),
            pl.BlockSpec((1, EMB), lambda i: (0, 0)),
            pl.BlockSpec((1, EMB), lambda i: (0, 0)),
            pl.BlockSpec((BS, 1), lambda i: (i, 0)),
            pl.BlockSpec((BS, 1), lambda i: (i, 0)),
        ],
        out_specs=[
            pl.BlockSpec((BS, EMB), lambda i: (i, 0)),
            pl.BlockSpec((BS, EMB), lambda i: (i, 0)),
        ],
        out_shape=[
            jax.ShapeDtypeStruct((N, EMB), jnp.float32),
            jax.ShapeDtypeStruct((N, EMB), jnp.float32),
        ],
    )(t, stats, scale_prev, bias_prev, Wl, bl, rootl, dis, invdeg)


def _accum_body(p0_ref, p1_ref, self_ref, dis_ref, t_ref, stats_ref, acc):
    i = pl.program_id(0)
    t = dis_ref[...] * (p0_ref[...] + p1_ref[...]) + self_ref[...]
    t_ref[...] = t

    @pl.when(i == 0)
    def _():
        acc[...] = jnp.zeros_like(acc)

    acc[0:1, :] += jnp.sum(t, axis=0, keepdims=True)
    acc[1:2, :] += jnp.sum(t * t, axis=0, keepdims=True)

    @pl.when(i == NBLK - 1)
    def _():
        stats_ref[...] = acc[...]


def _tc_accum(pflat, selfterm, dis):
    return pl.pallas_call(
        _accum_body,
        grid=(NBLK,),
        in_specs=[
            pl.BlockSpec((BS, EMB), lambda i: (i, 0)),
            pl.BlockSpec((BS, EMB), lambda i: (NBLK + i, 0)),
            pl.BlockSpec((BS, EMB), lambda i: (i, 0)),
            pl.BlockSpec((BS, 1), lambda i: (i, 0)),
        ],
        out_specs=[
            pl.BlockSpec((BS, EMB), lambda i: (i, 0)),
            pl.BlockSpec((2, EMB), lambda i: (0, 0)),
        ],
        out_shape=[
            jax.ShapeDtypeStruct((N, EMB), jnp.float32),
            jax.ShapeDtypeStruct((2, EMB), jnp.float32),
        ],
        scratch_shapes=[pltpu.VMEM((2, EMB), jnp.float32)],
    )(pflat, pflat, selfterm, dis)


def _final_body(t_ref, stats_ref, scale_ref, bias_ref, batch_ref,
                wp_ref, bp_ref, out_ref, pooled, counts):
    i = pl.program_id(0)
    h = _bn_apply(t_ref[...], stats_ref, scale_ref, bias_ref)
    gids = lax.broadcasted_iota(jnp.int32, (NGRAPH, BS), 0)
    onehot = jnp.where(gids == batch_ref[...], 1.0, 0.0).astype(jnp.float32)

    @pl.when(i == 0)
    def _():
        pooled[...] = jnp.zeros_like(pooled)
        counts[...] = jnp.zeros_like(counts)

    pooled[...] += jnp.dot(onehot, h, preferred_element_type=jnp.float32)
    counts[...] += jnp.sum(onehot, axis=1, keepdims=True)

    @pl.when(i == NBLK - 1)
    def _():
        hg = pooled[...] / jnp.maximum(counts[...], 1.0)
        out_ref[...] = jnp.dot(hg, wp_ref[...],
                               preferred_element_type=jnp.float32) + bp_ref[...]


def _tc_final(t, stats, scale4, bias4, batch_row, Wp, bp):
    return pl.pallas_call(
        _final_body,
        grid=(NBLK,),
        in_specs=[
            pl.BlockSpec((BS, EMB), lambda i: (i, 0)),
            pl.BlockSpec((2, EMB), lambda i: (0, 0)),
            pl.BlockSpec((1, EMB), lambda i: (0, 0)),
            pl.BlockSpec((1, EMB), lambda i: (0, 0)),
            pl.BlockSpec((1, BS), lambda i: (0, i)),
            pl.BlockSpec((EMB, NCLASS), lambda i: (0, 0)),
            pl.BlockSpec((1, NCLASS), lambda i: (0, 0)),
        ],
        out_specs=pl.BlockSpec((NGRAPH, NCLASS), lambda i: (0, 0)),
        out_shape=jax.ShapeDtypeStruct((NGRAPH, NCLASS), jnp.float32),
        scratch_shapes=[
            pltpu.VMEM((NGRAPH, EMB), jnp.float32),
            pltpu.VMEM((NGRAPH, 1), jnp.float32),
        ],
    )(t, stats, scale4, bias4, batch_row, Wp, bp)


def kernel(x, edge_index, batch, W, b, root, bn_scale, bn_bias, Wp, bp):
    src = edge_index[0]
    dst = edge_index[1]
    batch_row = batch.reshape(1, N)
    zeros128 = jnp.zeros((N, EMB), jnp.float32)
    zeros16 = jnp.zeros((N, 16), jnp.float32)
    ones16 = jnp.ones((WIN, 16), jnp.float32)

    hist = _sc_degree(src, ones16, zeros16)
    dis, invdeg = _tc_deg(hist)

    r2, selfterm = _tc_first(x, W[0], b[0].reshape(1, EMB),
                             root[0].reshape(1, EMB), dis, invdeg)
    t = stats = None
    for l in range(NLAYER):
        pflat = _sc_aggregate(r2, src, dst, zeros128)
        t, stats = _tc_accum(pflat, selfterm, dis)
        if l < NLAYER - 1:
            r2, selfterm = _tc_mid(
                t, stats,
                bn_scale[l].reshape(1, EMB), bn_bias[l].reshape(1, EMB),
                W[l + 1], b[l + 1].reshape(1, EMB),
                root[l + 1].reshape(1, EMB), dis, invdeg)

    return _tc_final(t, stats,
                     bn_scale[NLAYER - 1].reshape(1, EMB),
                     bn_bias[NLAYER - 1].reshape(1, EMB),
                     batch_row, Wp, bp.reshape(1, NCLASS))


# speedup vs baseline: 6.5903x; 6.5903x over previous
import functools

import jax
import jax.numpy as jnp
from jax import lax
from jax.experimental import pallas as pl
from jax.experimental.pallas import tpu as pltpu
from jax.experimental.pallas import tpu_sc as plsc

N = 10000
E = 320000
EMB = 128
NLAYER = 5
NGRAPH = 64
NCLASS = 128

NC = 2
NS = 16
NW = NC * NS
EW = E // NW
WIN = 80
NWIN = EW // WIN
RPT = 624
TAIL = N - NS * RPT
TAIL_OFF = NS * RPT

BS = 1000
NBLK = N // BS

_mesh = plsc.VectorSubcoreMesh(core_axis_name="c", subcore_axis_name="s",
                               num_cores=NC, num_subcores=NS)


@functools.partial(
    pl.kernel,
    out_type=jax.ShapeDtypeStruct((NC * N, EMB), jnp.float32),
    mesh=_mesh,
    scratch_types=[
        pltpu.VMEM((WIN,), jnp.int32),
        pltpu.VMEM((WIN, EMB), jnp.float32),
        pltpu.VMEM_SHARED((N, EMB), jnp.float32),
    ],
)
def _sc_degree(src_hbm, ones_hbm, zeros_hbm, out_hbm, sidx, ones_v, acc):
    c = lax.axis_index("c")
    s = lax.axis_index("s")
    pltpu.sync_copy(zeros_hbm.at[pl.ds(s * RPT, RPT)], acc.at[pl.ds(s * RPT, RPT)])

    @pl.when(s == NS - 1)
    def _():
        pltpu.sync_copy(zeros_hbm.at[pl.ds(TAIL_OFF, TAIL)],
                        acc.at[pl.ds(TAIL_OFF, TAIL)])

    pltpu.sync_copy(ones_hbm, ones_v)
    plsc.subcore_barrier()
    wid = c * NS + s

    @pl.loop(0, NWIN)
    def _(w):
        base = wid * EW + w * WIN
        pltpu.sync_copy(src_hbm.at[pl.ds(base, WIN)], sidx)
        pltpu.sync_copy(ones_v, acc.at[sidx], add=True)

    plsc.subcore_barrier()
    pltpu.sync_copy(acc.at[pl.ds(s * RPT, RPT)],
                    out_hbm.at[pl.ds(c * N + s * RPT, RPT)])

    @pl.when(s == NS - 1)
    def _():
        pltpu.sync_copy(acc.at[pl.ds(TAIL_OFF, TAIL)],
                        out_hbm.at[pl.ds(c * N + TAIL_OFF, TAIL)])


@functools.partial(
    pl.kernel,
    out_type=jax.ShapeDtypeStruct((NC * N, EMB), jnp.float32),
    mesh=_mesh,
    scratch_types=[
        pltpu.VMEM((WIN,), jnp.int32),
        pltpu.VMEM((WIN,), jnp.int32),
        pltpu.VMEM((WIN, EMB), jnp.float32),
        pltpu.VMEM_SHARED((N, EMB), jnp.float32),
        pltpu.SemaphoreType.DMA,
    ],
)
def _sc_aggregate(r2_hbm, src_hbm, dst_hbm, zeros_hbm, out_hbm,
                  sidx, didx, rows, acc, sem):
    c = lax.axis_index("c")
    s = lax.axis_index("s")
    pltpu.sync_copy(zeros_hbm.at[pl.ds(s * RPT, RPT)], acc.at[pl.ds(s * RPT, RPT)])

    @pl.when(s == NS - 1)
    def _():
        pltpu.sync_copy(zeros_hbm.at[pl.ds(TAIL_OFF, TAIL)],
                        acc.at[pl.ds(TAIL_OFF, TAIL)])

    plsc.subcore_barrier()
    wid = c * NS + s

    @pl.loop(0, NWIN)
    def _(w):
        base = wid * EW + w * WIN
        pltpu.sync_copy(src_hbm.at[pl.ds(base, WIN)], sidx)
        pltpu.sync_copy(dst_hbm.at[pl.ds(base, WIN)], didx)
        pltpu.async_copy(r2_hbm.at[sidx], rows, sem).wait()
        pltpu.sync_copy(rows, acc.at[didx], add=True)

    plsc.subcore_barrier()
    pltpu.sync_copy(acc.at[pl.ds(s * RPT, RPT)],
                    out_hbm.at[pl.ds(c * N + s * RPT, RPT)])

    @pl.when(s == NS - 1)
    def _():
        pltpu.sync_copy(acc.at[pl.ds(TAIL_OFF, TAIL)],
                        out_hbm.at[pl.ds(c * N + TAIL_OFF, TAIL)])


def _deg_body(p0_ref, p1_ref, dis_ref, inv_ref):
    deg = p0_ref[:, 0:1] + p1_ref[:, 0:1] + 1.0
    dis_ref[...] = lax.rsqrt(deg)
    inv_ref[...] = 1.0 / deg


def _tc_deg(hist):
    return pl.pallas_call(
        _deg_body,
        grid=(NBLK,),
        in_specs=[
            pl.BlockSpec((BS, EMB), lambda i: (i, 0)),
            pl.BlockSpec((BS, EMB), lambda i: (NBLK + i, 0)),
        ],
        out_specs=[
            pl.BlockSpec((BS, 1), lambda i: (i, 0)),
            pl.BlockSpec((BS, 1), lambda i: (i, 0)),
        ],
        out_shape=[
            jax.ShapeDtypeStruct((N, 1), jnp.float32),
            jax.ShapeDtypeStruct((N, 1), jnp.float32),
        ],
    )(hist, hist)


def _first_body(x_ref, w_ref, b_ref, root_ref, dis_ref, inv_ref,
                r2_ref, self_ref):
    hl = jnp.dot(x_ref[...], w_ref[...],
                 preferred_element_type=jnp.float32) + b_ref[...]
    r2_ref[...] = dis_ref[...] * jnp.maximum(hl, 0.0)
    self_ref[...] = jnp.maximum(hl + root_ref[...], 0.0) * inv_ref[...]


def _tc_first(x, Wl, bl, rootl, dis, invdeg):
    return pl.pallas_call(
        _first_body,
        grid=(NBLK,),
        in_specs=[
            pl.BlockSpec((BS, EMB), lambda i: (i, 0)),
            pl.BlockSpec((EMB, EMB), lambda i: (0, 0)),
            pl.BlockSpec((1, EMB), lambda i: (0, 0)),
            pl.BlockSpec((1, EMB), lambda i: (0, 0)),
            pl.BlockSpec((BS, 1), lambda i: (i, 0)),
            pl.BlockSpec((BS, 1), lambda i: (i, 0)),
        ],
        out_specs=[
            pl.BlockSpec((BS, EMB), lambda i: (i, 0)),
            pl.BlockSpec((BS, EMB), lambda i: (i, 0)),
        ],
        out_shape=[
            jax.ShapeDtypeStruct((N, EMB), jnp.float32),
            jax.ShapeDtypeStruct((N, EMB), jnp.float32),
        ],
    )(x, Wl, bl, rootl, dis, invdeg)


def _bn_apply(t, stats_ref, scale_ref, bias_ref):
    mean = stats_ref[0:1, :] * (1.0 / N)
    ex2 = stats_ref[1:2, :] * (1.0 / N)
    var = ex2 - mean * mean
    rstd = lax.rsqrt(var + 1e-5)
    return (t - mean) * rstd * scale_ref[...] + bias_ref[...]


def _mid_body(t_ref, stats_ref, scale_ref, bias_ref, w_ref, b_ref, root_ref,
              dis_ref, inv_ref, r2_ref, self_ref):
    h = jnp.maximum(_bn_apply(t_ref[...], stats_ref, scale_ref, bias_ref), 0.0)
    hl = jnp.dot(h, w_ref[...], preferred_element_type=jnp.float32) + b_ref[...]
    r2_ref[...] = dis_ref[...] * jnp.maximum(hl, 0.0)
    self_ref[...] = jnp.maximum(hl + root_ref[...], 0.0) * inv_ref[...]


def _tc_mid(t, stats, scale_prev, bias_prev, Wl, bl, rootl, dis, invdeg):
    return pl.pallas_call(
        _mid_body,
        grid=(NBLK,),
        in_specs=[
            pl.BlockSpec((BS, EMB), lambda i: (i, 0)),
            pl.BlockSpec((2, EMB), lambda i: (0, 0)),
            pl.BlockSpec((1, EMB), lambda i: (0, 0)),
            pl.BlockSpec((1, EMB), lambda i: (0, 0)),
            pl.BlockSpec((EMB, EMB), lambda i: (0, 0)),
            pl.BlockSpec((1, EMB), lambda i: (0, 0)),
            pl.BlockSpec((1, EMB), lambda i: (0, 0)),
            pl.BlockSpec((BS, 1), lambda i: (i, 0)),
            pl.BlockSpec((BS, 1), lambda i: (i, 0)),
        ],
        out_specs=[
            pl.BlockSpec((BS, EMB), lambda i: (i, 0)),
            pl.BlockSpec((BS, EMB), lambda i: (i, 0)),
        ],
        out_shape=[
            jax.ShapeDtypeStruct((N, EMB), jnp.float32),
            jax.ShapeDtypeStruct((N, EMB), jnp.float32),
        ],
    )(t, stats, scale_prev, bias_prev, Wl, bl, rootl, dis, invdeg)


def _accum_body(p0_ref, p1_ref, self_ref, dis_ref, t_ref, stats_ref, acc):
    i = pl.program_id(0)
    t = dis_ref[...] * (p0_ref[...] + p1_ref[...]) + self_ref[...]
    t_ref[...] = t

    @pl.when(i == 0)
    def _():
        acc[...] = jnp.zeros_like(acc)

    acc[0:1, :] += jnp.sum(t, axis=0, keepdims=True)
    acc[1:2, :] += jnp.sum(t * t, axis=0, keepdims=True)

    @pl.when(i == NBLK - 1)
    def _():
        stats_ref[...] = acc[...]


def _tc_accum(pflat, selfterm, dis):
    return pl.pallas_call(
        _accum_body,
        grid=(NBLK,),
        in_specs=[
            pl.BlockSpec((BS, EMB), lambda i: (i, 0)),
            pl.BlockSpec((BS, EMB), lambda i: (NBLK + i, 0)),
            pl.BlockSpec((BS, EMB), lambda i: (i, 0)),
            pl.BlockSpec((BS, 1), lambda i: (i, 0)),
        ],
        out_specs=[
            pl.BlockSpec((BS, EMB), lambda i: (i, 0)),
            pl.BlockSpec((2, EMB), lambda i: (0, 0)),
        ],
        out_shape=[
            jax.ShapeDtypeStruct((N, EMB), jnp.float32),
            jax.ShapeDtypeStruct((2, EMB), jnp.float32),
        ],
        scratch_shapes=[pltpu.VMEM((2, EMB), jnp.float32)],
    )(pflat, pflat, selfterm, dis)


def _final_body(t_ref, stats_ref, scale_ref, bias_ref, batch_ref,
                wp_ref, bp_ref, out_ref, pooled, counts):
    i = pl.program_id(0)
    h = _bn_apply(t_ref[...], stats_ref, scale_ref, bias_ref)
    gids = lax.broadcasted_iota(jnp.int32, (NGRAPH, BS), 0)
    onehot = jnp.where(gids == batch_ref[0], 1.0, 0.0).astype(jnp.float32)

    @pl.when(i == 0)
    def _():
        pooled[...] = jnp.zeros_like(pooled)
        counts[...] = jnp.zeros_like(counts)

    pooled[...] += jnp.dot(onehot, h, preferred_element_type=jnp.float32)
    counts[...] += jnp.sum(onehot, axis=1, keepdims=True)

    @pl.when(i == NBLK - 1)
    def _():
        hg = pooled[...] / jnp.maximum(counts[...], 1.0)
        out_ref[...] = jnp.dot(hg, wp_ref[...],
                               preferred_element_type=jnp.float32) + bp_ref[...]


def _tc_final(t, stats, scale4, bias4, batch_row, Wp, bp):
    return pl.pallas_call(
        _final_body,
        grid=(NBLK,),
        in_specs=[
            pl.BlockSpec((BS, EMB), lambda i: (i, 0)),
            pl.BlockSpec((2, EMB), lambda i: (0, 0)),
            pl.BlockSpec((1, EMB), lambda i: (0, 0)),
            pl.BlockSpec((1, EMB), lambda i: (0, 0)),
            pl.BlockSpec((1, 1, BS), lambda i: (i, 0, 0)),
            pl.BlockSpec((EMB, NCLASS), lambda i: (0, 0)),
            pl.BlockSpec((1, NCLASS), lambda i: (0, 0)),
        ],
        out_specs=pl.BlockSpec((NGRAPH, NCLASS), lambda i: (0, 0)),
        out_shape=jax.ShapeDtypeStruct((NGRAPH, NCLASS), jnp.float32),
        scratch_shapes=[
            pltpu.VMEM((NGRAPH, EMB), jnp.float32),
            pltpu.VMEM((NGRAPH, 1), jnp.float32),
        ],
    )(t, stats, scale4, bias4, batch_row, Wp, bp)


def kernel(x, edge_index, batch, W, b, root, bn_scale, bn_bias, Wp, bp):
    src = edge_index[0]
    dst = edge_index[1]
    batch_row = batch.reshape(NBLK, 1, BS)
    zeros128 = jnp.zeros((N, EMB), jnp.float32)
    ones_w = jnp.ones((WIN, EMB), jnp.float32)

    hist = _sc_degree(src, ones_w, zeros128)
    dis, invdeg = _tc_deg(hist)

    r2, selfterm = _tc_first(x, W[0], b[0].reshape(1, EMB),
                             root[0].reshape(1, EMB), dis, invdeg)
    t = stats = None
    for l in range(NLAYER):
        pflat = _sc_aggregate(r2, src, dst, zeros128)
        t, stats = _tc_accum(pflat, selfterm, dis)
        if l < NLAYER - 1:
            r2, selfterm = _tc_mid(
                t, stats,
                bn_scale[l].reshape(1, EMB), bn_bias[l].reshape(1, EMB),
                W[l + 1], b[l + 1].reshape(1, EMB),
                root[l + 1].reshape(1, EMB), dis, invdeg)

    return _tc_final(t, stats,
                     bn_scale[NLAYER - 1].reshape(1, EMB),
                     bn_bias[NLAYER - 1].reshape(1, EMB),
                     batch_row, Wp, bp.reshape(1, NCLASS))


# speedup vs baseline: 12.9318x; 1.9622x over previous
import functools

import jax
import jax.numpy as jnp
from jax import lax
from jax.experimental import pallas as pl
from jax.experimental.pallas import tpu as pltpu
from jax.experimental.pallas import tpu_sc as plsc

N = 10000
E = 320000
EMB = 128
NLAYER = 5
NGRAPH = 64
NCLASS = 128

NC = 2
NS = 16
NW = NC * NS
EW = E // NW
WIN = 125
NWIN = EW // WIN
CH = 16
NCH = NWIN // CH
RPT = 624
TAIL = N - NS * RPT
TAIL_OFF = NS * RPT

BS = 1000
NBLK = N // BS

_mesh = plsc.VectorSubcoreMesh(core_axis_name="c", subcore_axis_name="s",
                               num_cores=NC, num_subcores=NS)


@functools.partial(
    pl.kernel,
    out_type=jax.ShapeDtypeStruct((NC * N, EMB), jnp.float32),
    mesh=_mesh,
    scratch_types=[
        pltpu.VMEM((CH, WIN), jnp.int32),
        pltpu.VMEM((WIN, EMB), jnp.float32),
        pltpu.VMEM_SHARED((N, EMB), jnp.float32),
        pltpu.SemaphoreType.DMA,
    ],
)
def _sc_degree(src3_hbm, ones_hbm, zeros_hbm, out_hbm, sidx2, ones_v, acc, sem):
    c = lax.axis_index("c")
    s = lax.axis_index("s")
    pltpu.sync_copy(zeros_hbm.at[pl.ds(s * RPT, RPT)], acc.at[pl.ds(s * RPT, RPT)])

    @pl.when(s == NS - 1)
    def _():
        pltpu.sync_copy(zeros_hbm.at[pl.ds(TAIL_OFF, TAIL)],
                        acc.at[pl.ds(TAIL_OFF, TAIL)])

    pltpu.sync_copy(ones_hbm, ones_v)
    wid = c * NS + s
    plsc.subcore_barrier()

    @pl.loop(0, NCH)
    def _(ch):
        pltpu.sync_copy(src3_hbm.at[wid].at[pl.ds(ch * CH, CH)], sidx2)

        @pl.loop(0, CH, step=8)
        def _(w):
            for k in range(8):
                pltpu.async_copy(ones_v, acc.at[sidx2.at[w + k]], sem, add=True)
            for k in range(8):
                pltpu.make_async_copy(ones_v, acc.at[sidx2.at[w + k]], sem).wait()

    plsc.subcore_barrier()
    pltpu.sync_copy(acc.at[pl.ds(s * RPT, RPT)],
                    out_hbm.at[pl.ds(c * N + s * RPT, RPT)])

    @pl.when(s == NS - 1)
    def _():
        pltpu.sync_copy(acc.at[pl.ds(TAIL_OFF, TAIL)],
                        out_hbm.at[pl.ds(c * N + TAIL_OFF, TAIL)])


@functools.partial(
    pl.kernel,
    out_type=jax.ShapeDtypeStruct((NC * N, EMB), jnp.float32),
    mesh=_mesh,
    scratch_types=[
        pltpu.VMEM((CH, WIN), jnp.int32),
        pltpu.VMEM((CH, WIN), jnp.int32),
        pltpu.VMEM((WIN, EMB), jnp.float32),
        pltpu.VMEM((WIN, EMB), jnp.float32),
        pltpu.VMEM_SHARED((N, EMB), jnp.float32),
        pltpu.SemaphoreType.DMA,
        pltpu.SemaphoreType.DMA,
    ],
)
def _sc_aggregate(r2_hbm, src3_hbm, dst3_hbm, zeros_hbm, out_hbm,
                  sidx2, didx2, rows_a, rows_b, acc, gsem, ssem):
    c = lax.axis_index("c")
    s = lax.axis_index("s")
    pltpu.sync_copy(zeros_hbm.at[pl.ds(s * RPT, RPT)], acc.at[pl.ds(s * RPT, RPT)])

    @pl.when(s == NS - 1)
    def _():
        pltpu.sync_copy(zeros_hbm.at[pl.ds(TAIL_OFF, TAIL)],
                        acc.at[pl.ds(TAIL_OFF, TAIL)])

    wid = c * NS + s
    plsc.subcore_barrier()

    def gather(j, buf):
        return pltpu.async_copy(r2_hbm.at[sidx2.at[j]], buf, gsem)

    def gather_wait(j, buf):
        pltpu.make_async_copy(r2_hbm.at[sidx2.at[j]], buf, gsem).wait()

    def scatter(j, buf):
        return pltpu.async_copy(buf, acc.at[didx2.at[j]], ssem, add=True)

    @pl.loop(0, NCH)
    def _(ch):
        pltpu.sync_copy(src3_hbm.at[wid].at[pl.ds(ch * CH, CH)], sidx2)
        pltpu.sync_copy(dst3_hbm.at[wid].at[pl.ds(ch * CH, CH)], didx2)
        gather(0, rows_a)

        @pl.loop(0, CH - 2, step=2)
        def _(j):
            gather_wait(j, rows_a)
            gather(j + 1, rows_b)
            sc_a = scatter(j, rows_a)
            gather_wait(j + 1, rows_b)
            sc_a.wait()
            gather(j + 2, rows_a)
            sc_b = scatter(j + 1, rows_b)
            sc_b.wait()

        gather_wait(CH - 2, rows_a)
        gather(CH - 1, rows_b)
        sc_a = scatter(CH - 2, rows_a)
        gather_wait(CH - 1, rows_b)
        sc_a.wait()
        sc_b = scatter(CH - 1, rows_b)
        sc_b.wait()

    plsc.subcore_barrier()
    pltpu.sync_copy(acc.at[pl.ds(s * RPT, RPT)],
                    out_hbm.at[pl.ds(c * N + s * RPT, RPT)])

    @pl.when(s == NS - 1)
    def _():
        pltpu.sync_copy(acc.at[pl.ds(TAIL_OFF, TAIL)],
                        out_hbm.at[pl.ds(c * N + TAIL_OFF, TAIL)])


def _deg_body(p0_ref, p1_ref, dis_ref, inv_ref):
    deg = p0_ref[:, 0:1] + p1_ref[:, 0:1] + 1.0
    dis_ref[...] = lax.rsqrt(deg)
    inv_ref[...] = 1.0 / deg


def _tc_deg(hist):
    return pl.pallas_call(
        _deg_body,
        grid=(NBLK,),
        in_specs=[
            pl.BlockSpec((BS, EMB), lambda i: (i, 0)),
            pl.BlockSpec((BS, EMB), lambda i: (NBLK + i, 0)),
        ],
        out_specs=[
            pl.BlockSpec((BS, 1), lambda i: (i, 0)),
            pl.BlockSpec((BS, 1), lambda i: (i, 0)),
        ],
        out_shape=[
            jax.ShapeDtypeStruct((N, 1), jnp.float32),
            jax.ShapeDtypeStruct((N, 1), jnp.float32),
        ],
    )(hist, hist)


def _first_body(x_ref, w_ref, b_ref, root_ref, dis_ref, inv_ref,
                r2_ref, self_ref):
    hl = jnp.dot(x_ref[...], w_ref[...],
                 preferred_element_type=jnp.float32) + b_ref[...]
    r2_ref[...] = dis_ref[...] * jnp.maximum(hl, 0.0)
    self_ref[...] = jnp.maximum(hl + root_ref[...], 0.0) * inv_ref[...]


def _tc_first(x, Wl, bl, rootl, dis, invdeg):
    return pl.pallas_call(
        _first_body,
        grid=(NBLK,),
        in_specs=[
            pl.BlockSpec((BS, EMB), lambda i: (i, 0)),
            pl.BlockSpec((EMB, EMB), lambda i: (0, 0)),
            pl.BlockSpec((1, EMB), lambda i: (0, 0)),
            pl.BlockSpec((1, EMB), lambda i: (0, 0)),
            pl.BlockSpec((BS, 1), lambda i: (i, 0)),
            pl.BlockSpec((BS, 1), lambda i: (i, 0)),
        ],
        out_specs=[
            pl.BlockSpec((BS, EMB), lambda i: (i, 0)),
            pl.BlockSpec((BS, EMB), lambda i: (i, 0)),
        ],
        out_shape=[
            jax.ShapeDtypeStruct((N, EMB), jnp.float32),
            jax.ShapeDtypeStruct((N, EMB), jnp.float32),
        ],
    )(x, Wl, bl, rootl, dis, invdeg)


def _bn_apply(t, stats_ref, scale_ref, bias_ref):
    mean = stats_ref[0:1, :] * (1.0 / N)
    ex2 = stats_ref[1:2, :] * (1.0 / N)
    var = ex2 - mean * mean
    rstd = lax.rsqrt(var + 1e-5)
    return (t - mean) * rstd * scale_ref[...] + bias_ref[...]


def _mid_body(t_ref, stats_ref, scale_ref, bias_ref, w_ref, b_ref, root_ref,
              dis_ref, inv_ref, r2_ref, self_ref):
    h = jnp.maximum(_bn_apply(t_ref[...], stats_ref, scale_ref, bias_ref), 0.0)
    hl = jnp.dot(h, w_ref[...], preferred_element_type=jnp.float32) + b_ref[...]
    r2_ref[...] = dis_ref[...] * jnp.maximum(hl, 0.0)
    self_ref[...] = jnp.maximum(hl + root_ref[...], 0.0) * inv_ref[...]


def _tc_mid(t, stats, scale_prev, bias_prev, Wl, bl, rootl, dis, invdeg):
    return pl.pallas_call(
        _mid_body,
        grid=(NBLK,),
        in_specs=[
            pl.BlockSpec((BS, EMB), lambda i: (i, 0)),
            pl.BlockSpec((2, EMB), lambda i: (0, 0)),
            pl.BlockSpec((1, EMB), lambda i: (0, 0)),
            pl.BlockSpec((1, EMB), lambda i: (0, 0)),
            pl.BlockSpec((EMB, EMB), lambda i: (0, 0)),
            pl.BlockSpec((1, EMB), lambda i: (0, 0)),
            pl.BlockSpec((1, EMB), lambda i: (0, 0)),
            pl.BlockSpec((BS, 1), lambda i: (i, 0)),
            pl.BlockSpec((BS, 1), lambda i: (i, 0)),
        ],
        out_specs=[
            pl.BlockSpec((BS, EMB), lambda i: (i, 0)),
            pl.BlockSpec((BS, EMB), lambda i: (i, 0)),
        ],
        out_shape=[
            jax.ShapeDtypeStruct((N, EMB), jnp.float32),
            jax.ShapeDtypeStruct((N, EMB), jnp.float32),
        ],
    )(t, stats, scale_prev, bias_prev, Wl, bl, rootl, dis, invdeg)


def _accum_body(p0_ref, p1_ref, self_ref, dis_ref, t_ref, stats_ref, acc):
    i = pl.program_id(0)
    t = dis_ref[...] * (p0_ref[...] + p1_ref[...]) + self_ref[...]
    t_ref[...] = t

    @pl.when(i == 0)
    def _():
        acc[...] = jnp.zeros_like(acc)

    acc[0:1, :] += jnp.sum(t, axis=0, keepdims=True)
    acc[1:2, :] += jnp.sum(t * t, axis=0, keepdims=True)

    @pl.when(i == NBLK - 1)
    def _():
        stats_ref[...] = acc[...]


def _tc_accum(pflat, selfterm, dis):
    return pl.pallas_call(
        _accum_body,
        grid=(NBLK,),
        in_specs=[
            pl.BlockSpec((BS, EMB), lambda i: (i, 0)),
            pl.BlockSpec((BS, EMB), lambda i: (NBLK + i, 0)),
            pl.BlockSpec((BS, EMB), lambda i: (i, 0)),
            pl.BlockSpec((BS, 1), lambda i: (i, 0)),
        ],
        out_specs=[
            pl.BlockSpec((BS, EMB), lambda i: (i, 0)),
            pl.BlockSpec((2, EMB), lambda i: (0, 0)),
        ],
        out_shape=[
            jax.ShapeDtypeStruct((N, EMB), jnp.float32),
            jax.ShapeDtypeStruct((2, EMB), jnp.float32),
        ],
        scratch_shapes=[pltpu.VMEM((2, EMB), jnp.float32)],
    )(pflat, pflat, selfterm, dis)


def _final_body(t_ref, stats_ref, scale_ref, bias_ref, batch_ref,
                wp_ref, bp_ref, out_ref, pooled, counts):
    i = pl.program_id(0)
    h = _bn_apply(t_ref[...], stats_ref, scale_ref, bias_ref)
    gids = lax.broadcasted_iota(jnp.int32, (NGRAPH, BS), 0)
    onehot = jnp.where(gids == batch_ref[0], 1.0, 0.0).astype(jnp.float32)

    @pl.when(i == 0)
    def _():
        pooled[...] = jnp.zeros_like(pooled)
        counts[...] = jnp.zeros_like(counts)

    pooled[...] += jnp.dot(onehot, h, preferred_element_type=jnp.float32)
    counts[...] += jnp.sum(onehot, axis=1, keepdims=True)

    @pl.when(i == NBLK - 1)
    def _():
        hg = pooled[...] / jnp.maximum(counts[...], 1.0)
        out_ref[...] = jnp.dot(hg, wp_ref[...],
                               preferred_element_type=jnp.float32) + bp_ref[...]


def _tc_final(t, stats, scale4, bias4, batch_row, Wp, bp):
    return pl.pallas_call(
        _final_body,
        grid=(NBLK,),
        in_specs=[
            pl.BlockSpec((BS, EMB), lambda i: (i, 0)),
            pl.BlockSpec((2, EMB), lambda i: (0, 0)),
            pl.BlockSpec((1, EMB), lambda i: (0, 0)),
            pl.BlockSpec((1, EMB), lambda i: (0, 0)),
            pl.BlockSpec((1, 1, BS), lambda i: (i, 0, 0)),
            pl.BlockSpec((EMB, NCLASS), lambda i: (0, 0)),
            pl.BlockSpec((1, NCLASS), lambda i: (0, 0)),
        ],
        out_specs=pl.BlockSpec((NGRAPH, NCLASS), lambda i: (0, 0)),
        out_shape=jax.ShapeDtypeStruct((NGRAPH, NCLASS), jnp.float32),
        scratch_shapes=[
            pltpu.VMEM((NGRAPH, EMB), jnp.float32),
            pltpu.VMEM((NGRAPH, 1), jnp.float32),
        ],
    )(t, stats, scale4, bias4, batch_row, Wp, bp)


def kernel(x, edge_index, batch, W, b, root, bn_scale, bn_bias, Wp, bp):
    src3 = edge_index[0].reshape(NW, NWIN, WIN)
    dst3 = edge_index[1].reshape(NW, NWIN, WIN)
    batch_row = batch.reshape(NBLK, 1, BS)
    zeros128 = jnp.zeros((N, EMB), jnp.float32)
    ones_w = jnp.ones((WIN, EMB), jnp.float32)

    hist = _sc_degree(src3, ones_w, zeros128)
    dis, invdeg = _tc_deg(hist)

    r2, selfterm = _tc_first(x, W[0], b[0].reshape(1, EMB),
                             root[0].reshape(1, EMB), dis, invdeg)
    t = stats = None
    for l in range(NLAYER):
        pflat = _sc_aggregate(r2, src3, dst3, zeros128)
        t, stats = _tc_accum(pflat, selfterm, dis)
        if l < NLAYER - 1:
            r2, selfterm = _tc_mid(
                t, stats,
                bn_scale[l].reshape(1, EMB), bn_bias[l].reshape(1, EMB),
                W[l + 1], b[l + 1].reshape(1, EMB),
                root[l + 1].reshape(1, EMB), dis, invdeg)

    return _tc_final(t, stats,
                     bn_scale[NLAYER - 1].reshape(1, EMB),
                     bn_bias[NLAYER - 1].reshape(1, EMB),
                     batch_row, Wp, bp.reshape(1, NCLASS))


# speedup vs baseline: 15.6695x; 1.2117x over previous
import functools

import jax
import jax.numpy as jnp
from jax import lax
from jax.experimental import pallas as pl
from jax.experimental.pallas import tpu as pltpu
from jax.experimental.pallas import tpu_sc as plsc

N = 10000
E = 320000
EMB = 128
NLAYER = 5
NGRAPH = 64
NCLASS = 128

NC = 2
NS = 16
NW = NC * NS
EW = E // NW
WIN = 100
NWIN = EW // WIN
CH = 20
NCH = NWIN // CH
RPT = 624
TAIL = N - NS * RPT
TAIL_OFF = NS * RPT

BS = 1000
NBLK = N // BS

_mesh = plsc.VectorSubcoreMesh(core_axis_name="c", subcore_axis_name="s",
                               num_cores=NC, num_subcores=NS)


@functools.partial(
    pl.kernel,
    out_type=jax.ShapeDtypeStruct((NC * N, EMB), jnp.float32),
    mesh=_mesh,
    scratch_types=[
        pltpu.VMEM((CH, WIN), jnp.int32),
        pltpu.VMEM((WIN, EMB), jnp.float32),
        pltpu.VMEM_SHARED((N, EMB), jnp.float32),
        pltpu.SemaphoreType.DMA,
    ],
)
def _sc_degree(src3_hbm, ones_hbm, zeros_hbm, out_hbm, sidx2, ones_v, acc, sem):
    c = lax.axis_index("c")
    s = lax.axis_index("s")
    pltpu.sync_copy(zeros_hbm.at[pl.ds(s * RPT, RPT)], acc.at[pl.ds(s * RPT, RPT)])

    @pl.when(s == NS - 1)
    def _():
        pltpu.sync_copy(zeros_hbm.at[pl.ds(TAIL_OFF, TAIL)],
                        acc.at[pl.ds(TAIL_OFF, TAIL)])

    pltpu.sync_copy(ones_hbm, ones_v)
    wid = c * NS + s
    plsc.subcore_barrier()

    @pl.loop(0, NCH)
    def _(ch):
        pltpu.sync_copy(src3_hbm.at[wid].at[ch], sidx2)

        @pl.loop(0, CH, step=4)
        def _(w):
            for k in range(4):
                pltpu.async_copy(ones_v, acc.at[sidx2.at[w + k]], sem, add=True)
            for k in range(4):
                pltpu.make_async_copy(ones_v, acc.at[sidx2.at[w + k]], sem).wait()

    plsc.subcore_barrier()
    pltpu.sync_copy(acc.at[pl.ds(s * RPT, RPT)],
                    out_hbm.at[pl.ds(c * N + s * RPT, RPT)])

    @pl.when(s == NS - 1)
    def _():
        pltpu.sync_copy(acc.at[pl.ds(TAIL_OFF, TAIL)],
                        out_hbm.at[pl.ds(c * N + TAIL_OFF, TAIL)])


@functools.partial(
    pl.kernel,
    out_type=jax.ShapeDtypeStruct((NC * N, EMB), jnp.float32),
    mesh=_mesh,
    scratch_types=[
        pltpu.VMEM((CH, WIN), jnp.int32),
        pltpu.VMEM((CH, WIN), jnp.int32),
        pltpu.VMEM((WIN, EMB), jnp.float32),
        pltpu.VMEM((WIN, EMB), jnp.float32),
        pltpu.VMEM((WIN, EMB), jnp.float32),
        pltpu.VMEM_SHARED((N, EMB), jnp.float32),
        pltpu.SemaphoreType.DMA,
        pltpu.SemaphoreType.DMA,
    ],
)
def _sc_aggregate(r2_hbm, src3_hbm, dst3_hbm, zeros_hbm, out_hbm,
                  sidx2, didx2, rows_a, rows_b, rows_c, acc, gsem, ssem):
    c = lax.axis_index("c")
    s = lax.axis_index("s")
    pltpu.sync_copy(zeros_hbm.at[pl.ds(s * RPT, RPT)], acc.at[pl.ds(s * RPT, RPT)])

    @pl.when(s == NS - 1)
    def _():
        pltpu.sync_copy(zeros_hbm.at[pl.ds(TAIL_OFF, TAIL)],
                        acc.at[pl.ds(TAIL_OFF, TAIL)])

    wid = c * NS + s
    plsc.subcore_barrier()

    def gather(j, buf):
        return pltpu.async_copy(r2_hbm.at[sidx2.at[j]], buf, gsem)

    def gather_wait(j, buf):
        pltpu.make_async_copy(r2_hbm.at[sidx2.at[j]], buf, gsem).wait()

    def scatter(j, buf):
        return pltpu.async_copy(buf, acc.at[didx2.at[j]], ssem, add=True)

    rings = (rows_a, rows_b, rows_c)

    @pl.loop(0, NCH)
    def _(ch):
        pltpu.sync_copy(src3_hbm.at[wid].at[ch], sidx2)
        pltpu.sync_copy(dst3_hbm.at[wid].at[ch], didx2)
        gather(0, rows_a)
        gather(1, rows_b)
        gather(2, rows_c)

        @pl.loop(0, CH - 5, step=3)
        def _(j):
            for k in range(3):
                buf = rings[k]
                gather_wait(j + k, buf)
                sc = scatter(j + k, buf)
                sc.wait()
                gather(j + k + 3, buf)

        for jj in range(CH - 5, CH):
            buf = rings[jj % 3]
            gather_wait(jj, buf)
            sc = scatter(jj, buf)
            sc.wait()
            if jj + 3 < CH:
                gather(jj + 3, buf)

    plsc.subcore_barrier()
    pltpu.sync_copy(acc.at[pl.ds(s * RPT, RPT)],
                    out_hbm.at[pl.ds(c * N + s * RPT, RPT)])

    @pl.when(s == NS - 1)
    def _():
        pltpu.sync_copy(acc.at[pl.ds(TAIL_OFF, TAIL)],
                        out_hbm.at[pl.ds(c * N + TAIL_OFF, TAIL)])


def _deg_body(p0_ref, p1_ref, dis_ref, inv_ref):
    deg = p0_ref[:, 0:1] + p1_ref[:, 0:1] + 1.0
    dis_ref[...] = lax.rsqrt(deg)
    inv_ref[...] = 1.0 / deg


def _tc_deg(hist):
    return pl.pallas_call(
        _deg_body,
        grid=(NBLK,),
        in_specs=[
            pl.BlockSpec((BS, EMB), lambda i: (i, 0)),
            pl.BlockSpec((BS, EMB), lambda i: (NBLK + i, 0)),
        ],
        out_specs=[
            pl.BlockSpec((BS, 1), lambda i: (i, 0)),
            pl.BlockSpec((BS, 1), lambda i: (i, 0)),
        ],
        out_shape=[
            jax.ShapeDtypeStruct((N, 1), jnp.float32),
            jax.ShapeDtypeStruct((N, 1), jnp.float32),
        ],
    )(hist, hist)


def _first_body(x_ref, w_ref, b_ref, root_ref, dis_ref, inv_ref,
                r2_ref, self_ref):
    hl = jnp.dot(x_ref[...], w_ref[...],
                 preferred_element_type=jnp.float32) + b_ref[...]
    r2_ref[...] = dis_ref[...] * jnp.maximum(hl, 0.0)
    self_ref[...] = jnp.maximum(hl + root_ref[...], 0.0) * inv_ref[...]


def _tc_first(x, Wl, bl, rootl, dis, invdeg):
    return pl.pallas_call(
        _first_body,
        grid=(NBLK,),
        in_specs=[
            pl.BlockSpec((BS, EMB), lambda i: (i, 0)),
            pl.BlockSpec((EMB, EMB), lambda i: (0, 0)),
            pl.BlockSpec((1, EMB), lambda i: (0, 0)),
            pl.BlockSpec((1, EMB), lambda i: (0, 0)),
            pl.BlockSpec((BS, 1), lambda i: (i, 0)),
            pl.BlockSpec((BS, 1), lambda i: (i, 0)),
        ],
        out_specs=[
            pl.BlockSpec((BS, EMB), lambda i: (i, 0)),
            pl.BlockSpec((BS, EMB), lambda i: (i, 0)),
        ],
        out_shape=[
            jax.ShapeDtypeStruct((N, EMB), jnp.float32),
            jax.ShapeDtypeStruct((N, EMB), jnp.float32),
        ],
    )(x, Wl, bl, rootl, dis, invdeg)


def _bn_apply(t, stats_ref, scale_ref, bias_ref):
    mean = stats_ref[0:1, :] * (1.0 / N)
    ex2 = stats_ref[1:2, :] * (1.0 / N)
    var = ex2 - mean * mean
    rstd = lax.rsqrt(var + 1e-5)
    return (t - mean) * rstd * scale_ref[...] + bias_ref[...]


def _mid_body(t_ref, stats_ref, scale_ref, bias_ref, w_ref, b_ref, root_ref,
              dis_ref, inv_ref, r2_ref, self_ref):
    h = jnp.maximum(_bn_apply(t_ref[...], stats_ref, scale_ref, bias_ref), 0.0)
    hl = jnp.dot(h, w_ref[...], preferred_element_type=jnp.float32) + b_ref[...]
    r2_ref[...] = dis_ref[...] * jnp.maximum(hl, 0.0)
    self_ref[...] = jnp.maximum(hl + root_ref[...], 0.0) * inv_ref[...]


def _tc_mid(t, stats, scale_prev, bias_prev, Wl, bl, rootl, dis, invdeg):
    return pl.pallas_call(
        _mid_body,
        grid=(NBLK,),
        in_specs=[
            pl.BlockSpec((BS, EMB), lambda i: (i, 0)),
            pl.BlockSpec((2, EMB), lambda i: (0, 0)),
            pl.BlockSpec((1, EMB), lambda i: (0, 0)),
            pl.BlockSpec((1, EMB), lambda i: (0, 0)),
            pl.BlockSpec((EMB, EMB), lambda i: (0, 0)),
            pl.BlockSpec((1, EMB), lambda i: (0, 0)),
            pl.BlockSpec((1, EMB), lambda i: (0, 0)),
            pl.BlockSpec((BS, 1), lambda i: (i, 0)),
            pl.BlockSpec((BS, 1), lambda i: (i, 0)),
        ],
        out_specs=[
            pl.BlockSpec((BS, EMB), lambda i: (i, 0)),
            pl.BlockSpec((BS, EMB), lambda i: (i, 0)),
        ],
        out_shape=[
            jax.ShapeDtypeStruct((N, EMB), jnp.float32),
            jax.ShapeDtypeStruct((N, EMB), jnp.float32),
        ],
    )(t, stats, scale_prev, bias_prev, Wl, bl, rootl, dis, invdeg)


def _accum_body(p0_ref, p1_ref, self_ref, dis_ref, t_ref, stats_ref, acc):
    i = pl.program_id(0)
    t = dis_ref[...] * (p0_ref[...] + p1_ref[...]) + self_ref[...]
    t_ref[...] = t

    @pl.when(i == 0)
    def _():
        acc[...] = jnp.zeros_like(acc)

    acc[0:1, :] += jnp.sum(t, axis=0, keepdims=True)
    acc[1:2, :] += jnp.sum(t * t, axis=0, keepdims=True)

    @pl.when(i == NBLK - 1)
    def _():
        stats_ref[...] = acc[...]


def _tc_accum(pflat, selfterm, dis):
    return pl.pallas_call(
        _accum_body,
        grid=(NBLK,),
        in_specs=[
            pl.BlockSpec((BS, EMB), lambda i: (i, 0)),
            pl.BlockSpec((BS, EMB), lambda i: (NBLK + i, 0)),
            pl.BlockSpec((BS, EMB), lambda i: (i, 0)),
            pl.BlockSpec((BS, 1), lambda i: (i, 0)),
        ],
        out_specs=[
            pl.BlockSpec((BS, EMB), lambda i: (i, 0)),
            pl.BlockSpec((2, EMB), lambda i: (0, 0)),
        ],
        out_shape=[
            jax.ShapeDtypeStruct((N, EMB), jnp.float32),
            jax.ShapeDtypeStruct((2, EMB), jnp.float32),
        ],
        scratch_shapes=[pltpu.VMEM((2, EMB), jnp.float32)],
    )(pflat, pflat, selfterm, dis)


def _final_body(t_ref, stats_ref, scale_ref, bias_ref, batch_ref,
                wp_ref, bp_ref, out_ref, pooled, counts):
    i = pl.program_id(0)
    h = _bn_apply(t_ref[...], stats_ref, scale_ref, bias_ref)
    gids = lax.broadcasted_iota(jnp.int32, (NGRAPH, BS), 0)
    onehot = jnp.where(gids == batch_ref[0], 1.0, 0.0).astype(jnp.float32)

    @pl.when(i == 0)
    def _():
        pooled[...] = jnp.zeros_like(pooled)
        counts[...] = jnp.zeros_like(counts)

    pooled[...] += jnp.dot(onehot, h, preferred_element_type=jnp.float32)
    counts[...] += jnp.sum(onehot, axis=1, keepdims=True)

    @pl.when(i == NBLK - 1)
    def _():
        hg = pooled[...] / jnp.maximum(counts[...], 1.0)
        out_ref[...] = jnp.dot(hg, wp_ref[...],
                               preferred_element_type=jnp.float32) + bp_ref[...]


def _tc_final(t, stats, scale4, bias4, batch_row, Wp, bp):
    return pl.pallas_call(
        _final_body,
        grid=(NBLK,),
        in_specs=[
            pl.BlockSpec((BS, EMB), lambda i: (i, 0)),
            pl.BlockSpec((2, EMB), lambda i: (0, 0)),
            pl.BlockSpec((1, EMB), lambda i: (0, 0)),
            pl.BlockSpec((1, EMB), lambda i: (0, 0)),
            pl.BlockSpec((1, 1, BS), lambda i: (i, 0, 0)),
            pl.BlockSpec((EMB, NCLASS), lambda i: (0, 0)),
            pl.BlockSpec((1, NCLASS), lambda i: (0, 0)),
        ],
        out_specs=pl.BlockSpec((NGRAPH, NCLASS), lambda i: (0, 0)),
        out_shape=jax.ShapeDtypeStruct((NGRAPH, NCLASS), jnp.float32),
        scratch_shapes=[
            pltpu.VMEM((NGRAPH, EMB), jnp.float32),
            pltpu.VMEM((NGRAPH, 1), jnp.float32),
        ],
    )(t, stats, scale4, bias4, batch_row, Wp, bp)


def kernel(x, edge_index, batch, W, b, root, bn_scale, bn_bias, Wp, bp):
    src3 = edge_index[0].reshape(NW, NCH, CH, WIN)
    dst3 = edge_index[1].reshape(NW, NCH, CH, WIN)
    batch_row = batch.reshape(NBLK, 1, BS)
    zeros128 = jnp.zeros((N, EMB), jnp.float32)
    ones_w = jnp.ones((WIN, EMB), jnp.float32)

    hist = _sc_degree(src3, ones_w, zeros128)
    dis, invdeg = _tc_deg(hist)

    r2, selfterm = _tc_first(x, W[0], b[0].reshape(1, EMB),
                             root[0].reshape(1, EMB), dis, invdeg)
    t = stats = None
    for l in range(NLAYER):
        pflat = _sc_aggregate(r2, src3, dst3, zeros128)
        t, stats = _tc_accum(pflat, selfterm, dis)
        if l < NLAYER - 1:
            r2, selfterm = _tc_mid(
                t, stats,
                bn_scale[l].reshape(1, EMB), bn_bias[l].reshape(1, EMB),
                W[l + 1], b[l + 1].reshape(1, EMB),
                root[l + 1].reshape(1, EMB), dis, invdeg)

    return _tc_final(t, stats,
                     bn_scale[NLAYER - 1].reshape(1, EMB),
                     bn_bias[NLAYER - 1].reshape(1, EMB),
                     batch_row, Wp, bp.reshape(1, NCLASS))


# speedup vs baseline: 16.2022x; 1.0340x over previous
import functools

import jax
import jax.numpy as jnp
from jax import lax
from jax.experimental import pallas as pl
from jax.experimental.pallas import tpu as pltpu
from jax.experimental.pallas import tpu_sc as plsc

N = 10000
E = 320000
EMB = 128
NLAYER = 5
NGRAPH = 64
NCLASS = 128

NC = 2
NS = 16
NW = NC * NS
EW = E // NW
WIN = 100
NWIN = EW // WIN
CH = 20
NCH = NWIN // CH
RPT = 624
TAIL = N - NS * RPT
TAIL_OFF = NS * RPT

BS = 1000
NBLK = N // BS

_mesh = plsc.VectorSubcoreMesh(core_axis_name="c", subcore_axis_name="s",
                               num_cores=NC, num_subcores=NS)


@functools.partial(
    pl.kernel,
    out_type=jax.ShapeDtypeStruct((NC * N, EMB), jnp.float32),
    mesh=_mesh,
    scratch_types=[
        pltpu.VMEM((CH, WIN), jnp.int32),
        pltpu.VMEM((WIN, EMB), jnp.float32),
        pltpu.VMEM_SHARED((N, EMB), jnp.float32),
        pltpu.SemaphoreType.DMA,
    ],
)
def _sc_degree(src3_hbm, ones_hbm, zeros_hbm, out_hbm, sidx2, ones_v, acc, sem):
    c = lax.axis_index("c")
    s = lax.axis_index("s")
    pltpu.sync_copy(zeros_hbm.at[pl.ds(s * RPT, RPT)], acc.at[pl.ds(s * RPT, RPT)])

    @pl.when(s == NS - 1)
    def _():
        pltpu.sync_copy(zeros_hbm.at[pl.ds(TAIL_OFF, TAIL)],
                        acc.at[pl.ds(TAIL_OFF, TAIL)])

    pltpu.sync_copy(ones_hbm, ones_v)
    wid = c * NS + s
    plsc.subcore_barrier()

    @pl.loop(0, NCH)
    def _(ch):
        pltpu.sync_copy(src3_hbm.at[wid].at[ch], sidx2)

        @pl.loop(0, CH, step=4)
        def _(w):
            for k in range(4):
                pltpu.async_copy(ones_v, acc.at[sidx2.at[w + k]], sem, add=True)
            for k in range(4):
                pltpu.make_async_copy(ones_v, acc.at[sidx2.at[w + k]], sem).wait()

    plsc.subcore_barrier()
    pltpu.sync_copy(acc.at[pl.ds(s * RPT, RPT)],
                    out_hbm.at[pl.ds(c * N + s * RPT, RPT)])

    @pl.when(s == NS - 1)
    def _():
        pltpu.sync_copy(acc.at[pl.ds(TAIL_OFF, TAIL)],
                        out_hbm.at[pl.ds(c * N + TAIL_OFF, TAIL)])


@functools.partial(
    pl.kernel,
    out_type=jax.ShapeDtypeStruct((NC * N, EMB), jnp.float32),
    mesh=_mesh,
    scratch_types=[
        pltpu.VMEM((CH, WIN), jnp.int32),
        pltpu.VMEM((CH, WIN), jnp.int32),
        pltpu.VMEM((WIN, EMB), jnp.float32),
        pltpu.VMEM((WIN, EMB), jnp.float32),
        pltpu.VMEM((WIN, EMB), jnp.float32),
        pltpu.VMEM_SHARED((N, EMB), jnp.float32),
        pltpu.SemaphoreType.DMA,
        pltpu.SemaphoreType.DMA,
    ],
)
def _sc_aggregate(r2_hbm, src3_hbm, dst3_hbm, zeros_hbm, out_hbm,
                  sidx2, didx2, rows_a, rows_b, rows_c, acc, gsem, ssem):
    c = lax.axis_index("c")
    s = lax.axis_index("s")
    pltpu.sync_copy(zeros_hbm.at[pl.ds(s * RPT, RPT)], acc.at[pl.ds(s * RPT, RPT)])

    @pl.when(s == NS - 1)
    def _():
        pltpu.sync_copy(zeros_hbm.at[pl.ds(TAIL_OFF, TAIL)],
                        acc.at[pl.ds(TAIL_OFF, TAIL)])

    wid = c * NS + s
    plsc.subcore_barrier()

    def gather(j, buf):
        return pltpu.async_copy(r2_hbm.at[sidx2.at[j]], buf, gsem)

    def gather_wait(j, buf):
        pltpu.make_async_copy(r2_hbm.at[sidx2.at[j]], buf, gsem).wait()

    def scatter(j, buf):
        return pltpu.async_copy(buf, acc.at[didx2.at[j]], ssem, add=True)

    rings = (rows_a, rows_b, rows_c)

    @pl.loop(0, NCH)
    def _(ch):
        pltpu.sync_copy(src3_hbm.at[wid].at[ch], sidx2)
        pltpu.sync_copy(dst3_hbm.at[wid].at[ch], didx2)
        gather(0, rows_a)
        gather(1, rows_b)
        gather(2, rows_c)

        @pl.loop(0, CH - 5, step=3)
        def _(j):
            for k in range(3):
                buf = rings[k]
                gather_wait(j + k, buf)
                sc = scatter(j + k, buf)
                sc.wait()
                gather(j + k + 3, buf)

        for jj in range(CH - 5, CH):
            buf = rings[jj % 3]
            gather_wait(jj, buf)
            sc = scatter(jj, buf)
            sc.wait()
            if jj + 3 < CH:
                gather(jj + 3, buf)

    plsc.subcore_barrier()
    pltpu.sync_copy(acc.at[pl.ds(s * RPT, RPT)],
                    out_hbm.at[pl.ds(c * N + s * RPT, RPT)])

    @pl.when(s == NS - 1)
    def _():
        pltpu.sync_copy(acc.at[pl.ds(TAIL_OFF, TAIL)],
                        out_hbm.at[pl.ds(c * N + TAIL_OFF, TAIL)])


def _first_body(h0_ref, h1_ref, x_ref, w_ref, b_ref, root_ref,
                r2_ref, self_ref, dis_ref, inv_ref):
    deg = h0_ref[:, 0:1] + h1_ref[:, 0:1] + 1.0
    dis = lax.rsqrt(deg)
    inv = 1.0 / deg
    dis_ref[...] = dis
    inv_ref[...] = inv
    hl = jnp.dot(x_ref[...], w_ref[...],
                 preferred_element_type=jnp.float32) + b_ref[...]
    r2_ref[...] = dis * jnp.maximum(hl, 0.0)
    self_ref[...] = jnp.maximum(hl + root_ref[...], 0.0) * inv


def _tc_first(hist, x, Wl, bl, rootl):
    return pl.pallas_call(
        _first_body,
        grid=(NBLK,),
        in_specs=[
            pl.BlockSpec((BS, EMB), lambda i: (i, 0)),
            pl.BlockSpec((BS, EMB), lambda i: (NBLK + i, 0)),
            pl.BlockSpec((BS, EMB), lambda i: (i, 0)),
            pl.BlockSpec((EMB, EMB), lambda i: (0, 0)),
            pl.BlockSpec((1, EMB), lambda i: (0, 0)),
            pl.BlockSpec((1, EMB), lambda i: (0, 0)),
        ],
        out_specs=[
            pl.BlockSpec((BS, EMB), lambda i: (i, 0)),
            pl.BlockSpec((BS, EMB), lambda i: (i, 0)),
            pl.BlockSpec((BS, 1), lambda i: (i, 0)),
            pl.BlockSpec((BS, 1), lambda i: (i, 0)),
        ],
        out_shape=[
            jax.ShapeDtypeStruct((N, EMB), jnp.float32),
            jax.ShapeDtypeStruct((N, EMB), jnp.float32),
            jax.ShapeDtypeStruct((N, 1), jnp.float32),
            jax.ShapeDtypeStruct((N, 1), jnp.float32),
        ],
    )(hist, hist, x, Wl, bl, rootl)


def _bn_apply(t, stats, scale_ref, bias_ref):
    mean = stats[0:1, :] * (1.0 / N)
    ex2 = stats[1:2, :] * (1.0 / N)
    var = ex2 - mean * mean
    rstd = lax.rsqrt(var + 1e-5)
    return (t - mean) * rstd * scale_ref[...] + bias_ref[...]


def _accum_phase(i, p0_ref, p1_ref, self_ref, dis_ref, t_buf, stats):
    t = dis_ref[...] * (p0_ref[...] + p1_ref[...]) + self_ref[...]
    t_buf[pl.ds(i * BS, BS), :] = t

    @pl.when(i == 0)
    def _():
        stats[...] = jnp.zeros_like(stats)

    stats[0:1, :] += jnp.sum(t, axis=0, keepdims=True)
    stats[1:2, :] += jnp.sum(t * t, axis=0, keepdims=True)


def _layer_body(p0_ref, p1_ref, self_ref, dis_ref, inv_ref, scale_ref,
                bias_ref, w_ref, b_ref, root_ref, r2_ref, self_out_ref,
                t_buf, stats):
    i = pl.program_id(0)

    @pl.when(i < NBLK)
    def _():
        _accum_phase(i, p0_ref, p1_ref, self_ref, dis_ref, t_buf, stats)

    @pl.when(i >= NBLK)
    def _():
        j = i - NBLK
        t = t_buf[pl.ds(j * BS, BS), :]
        h = jnp.maximum(_bn_apply(t, stats[...], scale_ref, bias_ref), 0.0)
        hl = jnp.dot(h, w_ref[...],
                     preferred_element_type=jnp.float32) + b_ref[...]
        r2_ref[...] = dis_ref[...] * jnp.maximum(hl, 0.0)
        self_out_ref[...] = jnp.maximum(hl + root_ref[...], 0.0) * inv_ref[...]


def _tc_layer(pflat, selfterm, dis, invdeg, scale_prev, bias_prev,
              Wl, bl, rootl):
    lo = lambda i: (jnp.minimum(i, NBLK - 1), 0)
    hi = lambda i: (NBLK + jnp.minimum(i, NBLK - 1), 0)
    ph2 = lambda i: (jnp.maximum(i - NBLK, 0), 0)
    both = lambda i: (jnp.where(i < NBLK, i, i - NBLK), 0)
    return pl.pallas_call(
        _layer_body,
        grid=(2 * NBLK,),
        in_specs=[
            pl.BlockSpec((BS, EMB), lo),
            pl.BlockSpec((BS, EMB), hi),
            pl.BlockSpec((BS, EMB), lo),
            pl.BlockSpec((BS, 1), both),
            pl.BlockSpec((BS, 1), ph2),
            pl.BlockSpec((1, EMB), lambda i: (0, 0)),
            pl.BlockSpec((1, EMB), lambda i: (0, 0)),
            pl.BlockSpec((EMB, EMB), lambda i: (0, 0)),
            pl.BlockSpec((1, EMB), lambda i: (0, 0)),
            pl.BlockSpec((1, EMB), lambda i: (0, 0)),
        ],
        out_specs=[
            pl.BlockSpec((BS, EMB), ph2),
            pl.BlockSpec((BS, EMB), ph2),
        ],
        out_shape=[
            jax.ShapeDtypeStruct((N, EMB), jnp.float32),
            jax.ShapeDtypeStruct((N, EMB), jnp.float32),
        ],
        scratch_shapes=[
            pltpu.VMEM((N, EMB), jnp.float32),
            pltpu.VMEM((2, EMB), jnp.float32),
        ],
    )(pflat, pflat, selfterm, dis, invdeg, scale_prev, bias_prev,
      Wl, bl, rootl)


def _tail_body(p0_ref, p1_ref, self_ref, dis_ref, scale_ref, bias_ref,
               batch_ref, wp_ref, bp_ref, out_ref, t_buf, stats,
               pooled, counts):
    i = pl.program_id(0)

    @pl.when(i < NBLK)
    def _():
        _accum_phase(i, p0_ref, p1_ref, self_ref, dis_ref, t_buf, stats)

    @pl.when(i >= NBLK)
    def _():
        j = i - NBLK
        t = t_buf[pl.ds(j * BS, BS), :]
        h = _bn_apply(t, stats[...], scale_ref, bias_ref)
        gids = lax.broadcasted_iota(jnp.int32, (NGRAPH, BS), 0)
        onehot = jnp.where(gids == batch_ref[0], 1.0, 0.0).astype(jnp.float32)

        @pl.when(j == 0)
        def _():
            pooled[...] = jnp.zeros_like(pooled)
            counts[...] = jnp.zeros_like(counts)

        pooled[...] += jnp.dot(onehot, h, preferred_element_type=jnp.float32)
        counts[...] += jnp.sum(onehot, axis=1, keepdims=True)

        @pl.when(j == NBLK - 1)
        def _():
            hg = pooled[...] / jnp.maximum(counts[...], 1.0)
            out_ref[...] = jnp.dot(
                hg, wp_ref[...],
                preferred_element_type=jnp.float32) + bp_ref[...]


def _tc_tail(pflat, selfterm, dis, scale4, bias4, batch_row, Wp, bp):
    lo = lambda i: (jnp.minimum(i, NBLK - 1), 0)
    hi = lambda i: (NBLK + jnp.minimum(i, NBLK - 1), 0)
    both = lambda i: (jnp.where(i < NBLK, i, i - NBLK), 0)
    return pl.pallas_call(
        _tail_body,
        grid=(2 * NBLK,),
        in_specs=[
            pl.BlockSpec((BS, EMB), lo),
            pl.BlockSpec((BS, EMB), hi),
            pl.BlockSpec((BS, EMB), lo),
            pl.BlockSpec((BS, 1), lo),
            pl.BlockSpec((1, EMB), lambda i: (0, 0)),
            pl.BlockSpec((1, EMB), lambda i: (0, 0)),
            pl.BlockSpec((1, 1, BS), lambda i: (jnp.maximum(i - NBLK, 0), 0, 0)),
            pl.BlockSpec((EMB, NCLASS), lambda i: (0, 0)),
            pl.BlockSpec((1, NCLASS), lambda i: (0, 0)),
        ],
        out_specs=pl.BlockSpec((NGRAPH, NCLASS), lambda i: (0, 0)),
        out_shape=jax.ShapeDtypeStruct((NGRAPH, NCLASS), jnp.float32),
        scratch_shapes=[
            pltpu.VMEM((N, EMB), jnp.float32),
            pltpu.VMEM((2, EMB), jnp.float32),
            pltpu.VMEM((NGRAPH, EMB), jnp.float32),
            pltpu.VMEM((NGRAPH, 1), jnp.float32),
        ],
    )(pflat, pflat, selfterm, dis, scale4, bias4, batch_row, Wp, bp)


def kernel(x, edge_index, batch, W, b, root, bn_scale, bn_bias, Wp, bp):
    src3 = edge_index[0].reshape(NW, NCH, CH, WIN)
    dst3 = edge_index[1].reshape(NW, NCH, CH, WIN)
    batch_row = batch.reshape(NBLK, 1, BS)
    zeros128 = jnp.zeros((N, EMB), jnp.float32)
    ones_w = jnp.ones((WIN, EMB), jnp.float32)

    hist = _sc_degree(src3, ones_w, zeros128)
    r2, selfterm, dis, invdeg = _tc_first(hist, x, W[0], b[0].reshape(1, EMB),
                                          root[0].reshape(1, EMB))
    for l in range(NLAYER - 1):
        pflat = _sc_aggregate(r2, src3, dst3, zeros128)
        r2, selfterm = _tc_layer(
            pflat, selfterm, dis, invdeg,
            bn_scale[l].reshape(1, EMB), bn_bias[l].reshape(1, EMB),
            W[l + 1], b[l + 1].reshape(1, EMB), root[l + 1].reshape(1, EMB))

    pflat = _sc_aggregate(r2, src3, dst3, zeros128)
    return _tc_tail(pflat, selfterm, dis,
                    bn_scale[NLAYER - 1].reshape(1, EMB),
                    bn_bias[NLAYER - 1].reshape(1, EMB),
                    batch_row, Wp, bp.reshape(1, NCLASS))


# speedup vs baseline: 17.1530x; 1.0587x over previous
import dataclasses
import functools

import jax
import jax.numpy as jnp
from jax import lax
from jax.experimental import pallas as pl
from jax.experimental.pallas import tpu as pltpu
from jax.experimental.pallas import tpu_sc as plsc

N = 10000
E = 320000
EMB = 128
NLAYER = 5
NGRAPH = 64
NCLASS = 128

NC = 2
NS = 16
NW = NC * NS
EW = E // NW
WIN = 100
NWIN = EW // WIN
CH = 20
NCH = NWIN // CH
RPT = 624
TAIL = N - NS * RPT
TAIL_OFF = NS * RPT

BS = 1000
NBLK = N // BS

_mesh = plsc.VectorSubcoreMesh(core_axis_name="c", subcore_axis_name="s",
                               num_cores=NC, num_subcores=NS)

_sc_params = pltpu.CompilerParams()
if "needs_layout_passes" in pltpu.CompilerParams.__dataclass_fields__:
    _sc_params = dataclasses.replace(_sc_params, needs_layout_passes=False)


@functools.partial(
    pl.kernel,
    out_type=jax.ShapeDtypeStruct((NBLK, NW, 1024), jnp.float32),
    mesh=_mesh,
    compiler_params=_sc_params,
    scratch_types=[
        pltpu.VMEM((EW,), jnp.int32),
        pltpu.VMEM((NBLK * 1024,), jnp.float32),
    ],
)
def _sc_degree(src_hbm, out_hbm, sidx, hist):
    c = lax.axis_index("c")
    s = lax.axis_index("s")
    wid = c * NS + s
    pltpu.sync_copy(src_hbm.at[pl.ds(wid * EW, EW)], sidx)
    zeros16 = jnp.zeros((16,), jnp.float32)

    @pl.loop(0, NBLK * 1024, step=16)
    def _(i):
        hist[pl.ds(i, 16)] = zeros16

    ones16 = jnp.ones((16,), jnp.float32)
    bs16 = jnp.full((16,), BS, jnp.int32)

    @pl.loop(0, EW, step=16)
    def _(i):
        idx = sidx[pl.ds(i, 16)]
        q = lax.div(idx, bs16)
        pos = idx + q * (1024 - BS)
        plsc.addupdate_scatter(hist, [pos], ones16)

    @pl.loop(0, NBLK)
    def _(ib):
        pltpu.sync_copy(hist.at[pl.ds(ib * 1024, 1024)], out_hbm.at[ib].at[wid])


@functools.partial(
    pl.kernel,
    out_type=jax.ShapeDtypeStruct((NC * N, EMB), jnp.float32),
    mesh=_mesh,
    scratch_types=[
        pltpu.VMEM((CH, WIN), jnp.int32),
        pltpu.VMEM((CH, WIN), jnp.int32),
        pltpu.VMEM((WIN, EMB), jnp.float32),
        pltpu.VMEM((WIN, EMB), jnp.float32),
        pltpu.VMEM((WIN, EMB), jnp.float32),
        pltpu.VMEM_SHARED((N, EMB), jnp.float32),
        pltpu.SemaphoreType.DMA,
        pltpu.SemaphoreType.DMA,
    ],
)
def _sc_aggregate(r2_hbm, src3_hbm, dst3_hbm, zeros_hbm, out_hbm,
                  sidx2, didx2, rows_a, rows_b, rows_c, acc, gsem, ssem):
    c = lax.axis_index("c")
    s = lax.axis_index("s")
    pltpu.sync_copy(zeros_hbm.at[pl.ds(s * RPT, RPT)], acc.at[pl.ds(s * RPT, RPT)])

    @pl.when(s == NS - 1)
    def _():
        pltpu.sync_copy(zeros_hbm.at[pl.ds(TAIL_OFF, TAIL)],
                        acc.at[pl.ds(TAIL_OFF, TAIL)])

    wid = c * NS + s
    plsc.subcore_barrier()

    def gather(j, buf):
        return pltpu.async_copy(r2_hbm.at[sidx2.at[j]], buf, gsem)

    def gather_wait(j, buf):
        pltpu.make_async_copy(r2_hbm.at[sidx2.at[j]], buf, gsem).wait()

    def scatter(j, buf):
        return pltpu.async_copy(buf, acc.at[didx2.at[j]], ssem, add=True)

    rings = (rows_a, rows_b, rows_c)

    @pl.loop(0, NCH)
    def _(ch):
        pltpu.sync_copy(src3_hbm.at[wid].at[ch], sidx2)
        pltpu.sync_copy(dst3_hbm.at[wid].at[ch], didx2)
        gather(0, rows_a)
        gather(1, rows_b)
        gather(2, rows_c)

        @pl.loop(0, CH - 5, step=3)
        def _(j):
            for k in range(3):
                buf = rings[k]
                gather_wait(j + k, buf)
                sc = scatter(j + k, buf)
                sc.wait()
                gather(j + k + 3, buf)

        for jj in range(CH - 5, CH):
            buf = rings[jj % 3]
            gather_wait(jj, buf)
            sc = scatter(jj, buf)
            sc.wait()
            if jj + 3 < CH:
                gather(jj + 3, buf)

    plsc.subcore_barrier()
    pltpu.sync_copy(acc.at[pl.ds(s * RPT, RPT)],
                    out_hbm.at[pl.ds(c * N + s * RPT, RPT)])

    @pl.when(s == NS - 1)
    def _():
        pltpu.sync_copy(acc.at[pl.ds(TAIL_OFF, TAIL)],
                        out_hbm.at[pl.ds(c * N + TAIL_OFF, TAIL)])


def _first_body(h32_ref, x_ref, w_ref, b_ref, root_ref,
                r2_ref, self_ref, dis_ref, inv_ref, eye):
    i = pl.program_id(0)

    @pl.when(i == 0)
    def _():
        r = lax.broadcasted_iota(jnp.int32, (BS, BS), 0)
        cc = lax.broadcasted_iota(jnp.int32, (BS, BS), 1)
        eye[...] = jnp.where(r == cc, 1.0, 0.0).astype(jnp.float32)

    deg_row = jnp.sum(h32_ref[0], axis=0, keepdims=True)[:, 0:BS] + 1.0
    deg = lax.dot_general(eye[...], deg_row, (((1,), (1,)), ((), ())),
                          preferred_element_type=jnp.float32)
    dis = lax.rsqrt(deg)
    inv = 1.0 / deg
    dis_ref[...] = dis
    inv_ref[...] = inv
    hl = jnp.dot(x_ref[...], w_ref[...],
                 preferred_element_type=jnp.float32) + b_ref[...]
    r2_ref[...] = dis * jnp.maximum(hl, 0.0)
    self_ref[...] = jnp.maximum(hl + root_ref[...], 0.0) * inv


def _tc_first(hist, x, Wl, bl, rootl):
    return pl.pallas_call(
        _first_body,
        grid=(NBLK,),
        in_specs=[
            pl.BlockSpec((1, NW, 1024), lambda i: (i, 0, 0)),
            pl.BlockSpec((BS, EMB), lambda i: (i, 0)),
            pl.BlockSpec((EMB, EMB), lambda i: (0, 0)),
            pl.BlockSpec((1, EMB), lambda i: (0, 0)),
            pl.BlockSpec((1, EMB), lambda i: (0, 0)),
        ],
        out_specs=[
            pl.BlockSpec((BS, EMB), lambda i: (i, 0)),
            pl.BlockSpec((BS, EMB), lambda i: (i, 0)),
            pl.BlockSpec((BS, 1), lambda i: (i, 0)),
            pl.BlockSpec((BS, 1), lambda i: (i, 0)),
        ],
        out_shape=[
            jax.ShapeDtypeStruct((N, EMB), jnp.float32),
            jax.ShapeDtypeStruct((N, EMB), jnp.float32),
            jax.ShapeDtypeStruct((N, 1), jnp.float32),
            jax.ShapeDtypeStruct((N, 1), jnp.float32),
        ],
        scratch_shapes=[pltpu.VMEM((BS, BS), jnp.float32)],
    )(hist, x, Wl, bl, rootl)


def _bn_apply(t, stats, scale_ref, bias_ref):
    mean = stats[0:1, :] * (1.0 / N)
    ex2 = stats[1:2, :] * (1.0 / N)
    var = ex2 - mean * mean
    rstd = lax.rsqrt(var + 1e-5)
    return (t - mean) * rstd * scale_ref[...] + bias_ref[...]


def _accum_phase(i, p0_ref, p1_ref, self_ref, dis_ref, t_buf, stats):
    t = dis_ref[...] * (p0_ref[...] + p1_ref[...]) + self_ref[...]
    t_buf[pl.ds(i * BS, BS), :] = t

    @pl.when(i == 0)
    def _():
        stats[...] = jnp.zeros_like(stats)

    stats[0:1, :] += jnp.sum(t, axis=0, keepdims=True)
    stats[1:2, :] += jnp.sum(t * t, axis=0, keepdims=True)


def _layer_body(p0_ref, p1_ref, self_ref, dis_ref, inv_ref, scale_ref,
                bias_ref, w_ref, b_ref, root_ref, r2_ref, self_out_ref,
                t_buf, stats):
    i = pl.program_id(0)

    @pl.when(i < NBLK)
    def _():
        _accum_phase(i, p0_ref, p1_ref, self_ref, dis_ref, t_buf, stats)

    @pl.when(i >= NBLK)
    def _():
        j = i - NBLK
        t = t_buf[pl.ds(j * BS, BS), :]
        h = jnp.maximum(_bn_apply(t, stats[...], scale_ref, bias_ref), 0.0)
        hl = jnp.dot(h, w_ref[...],
                     preferred_element_type=jnp.float32) + b_ref[...]
        r2_ref[...] = dis_ref[...] * jnp.maximum(hl, 0.0)
        self_out_ref[...] = jnp.maximum(hl + root_ref[...], 0.0) * inv_ref[...]


def _tc_layer(pflat, selfterm, dis, invdeg, scale_prev, bias_prev,
              Wl, bl, rootl):
    lo = lambda i: (jnp.minimum(i, NBLK - 1), 0)
    hi = lambda i: (NBLK + jnp.minimum(i, NBLK - 1), 0)
    ph2 = lambda i: (jnp.maximum(i - NBLK, 0), 0)
    both = lambda i: (jnp.where(i < NBLK, i, i - NBLK), 0)
    return pl.pallas_call(
        _layer_body,
        grid=(2 * NBLK,),
        in_specs=[
            pl.BlockSpec((BS, EMB), lo),
            pl.BlockSpec((BS, EMB), hi),
            pl.BlockSpec((BS, EMB), lo),
            pl.BlockSpec((BS, 1), both),
            pl.BlockSpec((BS, 1), ph2),
            pl.BlockSpec((1, EMB), lambda i: (0, 0)),
            pl.BlockSpec((1, EMB), lambda i: (0, 0)),
            pl.BlockSpec((EMB, EMB), lambda i: (0, 0)),
            pl.BlockSpec((1, EMB), lambda i: (0, 0)),
            pl.BlockSpec((1, EMB), lambda i: (0, 0)),
        ],
        out_specs=[
            pl.BlockSpec((BS, EMB), ph2),
            pl.BlockSpec((BS, EMB), ph2),
        ],
        out_shape=[
            jax.ShapeDtypeStruct((N, EMB), jnp.float32),
            jax.ShapeDtypeStruct((N, EMB), jnp.float32),
        ],
        scratch_shapes=[
            pltpu.VMEM((N, EMB), jnp.float32),
            pltpu.VMEM((2, EMB), jnp.float32),
        ],
    )(pflat, pflat, selfterm, dis, invdeg, scale_prev, bias_prev,
      Wl, bl, rootl)


def _tail_body(p0_ref, p1_ref, self_ref, dis_ref, scale_ref, bias_ref,
               batch_ref, wp_ref, bp_ref, out_ref, t_buf, stats,
               pooled, counts):
    i = pl.program_id(0)

    @pl.when(i < NBLK)
    def _():
        _accum_phase(i, p0_ref, p1_ref, self_ref, dis_ref, t_buf, stats)

    @pl.when(i >= NBLK)
    def _():
        j = i - NBLK
        t = t_buf[pl.ds(j * BS, BS), :]
        h = _bn_apply(t, stats[...], scale_ref, bias_ref)
        gids = lax.broadcasted_iota(jnp.int32, (NGRAPH, BS), 0)
        onehot = jnp.where(gids == batch_ref[0], 1.0, 0.0).astype(jnp.float32)

        @pl.when(j == 0)
        def _():
            pooled[...] = jnp.zeros_like(pooled)
            counts[...] = jnp.zeros_like(counts)

        pooled[...] += jnp.dot(onehot, h, preferred_element_type=jnp.float32)
        counts[...] += jnp.sum(onehot, axis=1, keepdims=True)

        @pl.when(j == NBLK - 1)
        def _():
            hg = pooled[...] / jnp.maximum(counts[...], 1.0)
            out_ref[...] = jnp.dot(
                hg, wp_ref[...],
                preferred_element_type=jnp.float32) + bp_ref[...]


def _tc_tail(pflat, selfterm, dis, scale4, bias4, batch_row, Wp, bp):
    lo = lambda i: (jnp.minimum(i, NBLK - 1), 0)
    hi = lambda i: (NBLK + jnp.minimum(i, NBLK - 1), 0)
    both = lambda i: (jnp.where(i < NBLK, i, i - NBLK), 0)
    return pl.pallas_call(
        _tail_body,
        grid=(2 * NBLK,),
        in_specs=[
            pl.BlockSpec((BS, EMB), lo),
            pl.BlockSpec((BS, EMB), hi),
            pl.BlockSpec((BS, EMB), lo),
            pl.BlockSpec((BS, 1), lo),
            pl.BlockSpec((1, EMB), lambda i: (0, 0)),
            pl.BlockSpec((1, EMB), lambda i: (0, 0)),
            pl.BlockSpec((1, 1, BS), lambda i: (jnp.maximum(i - NBLK, 0), 0, 0)),
            pl.BlockSpec((EMB, NCLASS), lambda i: (0, 0)),
            pl.BlockSpec((1, NCLASS), lambda i: (0, 0)),
        ],
        out_specs=pl.BlockSpec((NGRAPH, NCLASS), lambda i: (0, 0)),
        out_shape=jax.ShapeDtypeStruct((NGRAPH, NCLASS), jnp.float32),
        scratch_shapes=[
            pltpu.VMEM((N, EMB), jnp.float32),
            pltpu.VMEM((2, EMB), jnp.float32),
            pltpu.VMEM((NGRAPH, EMB), jnp.float32),
            pltpu.VMEM((NGRAPH, 1), jnp.float32),
        ],
    )(pflat, pflat, selfterm, dis, scale4, bias4, batch_row, Wp, bp)


def kernel(x, edge_index, batch, W, b, root, bn_scale, bn_bias, Wp, bp):
    src = edge_index[0]
    src3 = src.reshape(NW, NCH, CH, WIN)
    dst3 = edge_index[1].reshape(NW, NCH, CH, WIN)
    batch_row = batch.reshape(NBLK, 1, BS)
    zeros128 = jnp.zeros((N, EMB), jnp.float32)

    hist = _sc_degree(src)
    r2, selfterm, dis, invdeg = _tc_first(hist, x, W[0], b[0].reshape(1, EMB),
                                          root[0].reshape(1, EMB))
    for l in range(NLAYER - 1):
        pflat = _sc_aggregate(r2, src3, dst3, zeros128)
        r2, selfterm = _tc_layer(
            pflat, selfterm, dis, invdeg,
            bn_scale[l].reshape(1, EMB), bn_bias[l].reshape(1, EMB),
            W[l + 1], b[l + 1].reshape(1, EMB), root[l + 1].reshape(1, EMB))

    pflat = _sc_aggregate(r2, src3, dst3, zeros128)
    return _tc_tail(pflat, selfterm, dis,
                    bn_scale[NLAYER - 1].reshape(1, EMB),
                    bn_bias[NLAYER - 1].reshape(1, EMB),
                    batch_row, Wp, bp.reshape(1, NCLASS))


# speedup vs baseline: 17.2237x; 1.0041x over previous
import dataclasses
import functools

import jax
import jax.numpy as jnp
from jax import lax
from jax.experimental import pallas as pl
from jax.experimental.pallas import tpu as pltpu
from jax.experimental.pallas import tpu_sc as plsc

N = 10000
E = 320000
EMB = 128
NLAYER = 5
NGRAPH = 64
NCLASS = 128

NC = 2
NS = 16
NW = NC * NS
EW = E // NW
WIN = 100
NWIN = EW // WIN
CH = 20
NCH = NWIN // CH
RPT = 624
TAIL = N - NS * RPT
TAIL_OFF = NS * RPT

BS = 1000
NBLK = N // BS

_mesh = plsc.VectorSubcoreMesh(core_axis_name="c", subcore_axis_name="s",
                               num_cores=NC, num_subcores=NS)

_sc_params = pltpu.CompilerParams()
if "needs_layout_passes" in pltpu.CompilerParams.__dataclass_fields__:
    _sc_params = dataclasses.replace(_sc_params, needs_layout_passes=False)


@functools.partial(
    pl.kernel,
    out_type=jax.ShapeDtypeStruct((NBLK, NW, 1024), jnp.float32),
    mesh=_mesh,
    compiler_params=_sc_params,
    scratch_types=[
        pltpu.VMEM((EW,), jnp.int32),
        pltpu.VMEM((NBLK * 1024,), jnp.float32),
        pltpu.SemaphoreType.DMA,
    ],
)
def _sc_degree(src_hbm, out_hbm, sidx, hist, dsem):
    c = lax.axis_index("c")
    s = lax.axis_index("s")
    wid = c * NS + s
    pltpu.sync_copy(src_hbm.at[pl.ds(wid * EW, EW)], sidx)
    zeros16 = jnp.zeros((16,), jnp.float32)

    @pl.loop(0, NBLK * 1024, step=128)
    def _(i):
        for k in range(8):
            hist[pl.ds(i + 16 * k, 16)] = zeros16

    ones16 = jnp.ones((16,), jnp.float32)
    bs16 = jnp.full((16,), BS, jnp.int32)

    @pl.loop(0, EW, step=80)
    def _(i):
        for k in range(5):
            idx = sidx[pl.ds(i + 16 * k, 16)]
            q = lax.div(idx, bs16)
            pos = idx + q * (1024 - BS)
            plsc.addupdate_scatter(hist, [pos], ones16)

    for ib in range(NBLK):
        pltpu.async_copy(hist.at[pl.ds(ib * 1024, 1024)],
                         out_hbm.at[ib].at[wid], dsem)
    for ib in range(NBLK):
        pltpu.make_async_copy(hist.at[pl.ds(ib * 1024, 1024)],
                              out_hbm.at[ib].at[wid], dsem).wait()


@functools.partial(
    pl.kernel,
    out_type=jax.ShapeDtypeStruct((NC * N, EMB), jnp.float32),
    mesh=_mesh,
    scratch_types=[
        pltpu.VMEM((CH, WIN), jnp.int32),
        pltpu.VMEM((CH, WIN), jnp.int32),
        pltpu.VMEM((WIN, EMB), jnp.float32),
        pltpu.VMEM((WIN, EMB), jnp.float32),
        pltpu.VMEM((WIN, EMB), jnp.float32),
        pltpu.VMEM_SHARED((N, EMB), jnp.float32),
        pltpu.SemaphoreType.DMA,
        pltpu.SemaphoreType.DMA,
    ],
)
def _sc_aggregate(r2_hbm, src3_hbm, dst3_hbm, zeros_hbm, out_hbm,
                  sidx2, didx2, rows_a, rows_b, rows_c, acc, gsem, ssem):
    c = lax.axis_index("c")
    s = lax.axis_index("s")
    pltpu.sync_copy(zeros_hbm.at[pl.ds(s * RPT, RPT)], acc.at[pl.ds(s * RPT, RPT)])

    @pl.when(s == NS - 1)
    def _():
        pltpu.sync_copy(zeros_hbm.at[pl.ds(TAIL_OFF, TAIL)],
                        acc.at[pl.ds(TAIL_OFF, TAIL)])

    wid = c * NS + s
    plsc.subcore_barrier()

    def gather(j, buf):
        return pltpu.async_copy(r2_hbm.at[sidx2.at[j]], buf, gsem)

    def gather_wait(j, buf):
        pltpu.make_async_copy(r2_hbm.at[sidx2.at[j]], buf, gsem).wait()

    def scatter(j, buf):
        return pltpu.async_copy(buf, acc.at[didx2.at[j]], ssem, add=True)

    rings = (rows_a, rows_b, rows_c)

    @pl.loop(0, NCH)
    def _(ch):
        pltpu.sync_copy(src3_hbm.at[wid].at[ch], sidx2)
        pltpu.sync_copy(dst3_hbm.at[wid].at[ch], didx2)
        gather(0, rows_a)
        gather(1, rows_b)
        gather(2, rows_c)

        @pl.loop(0, CH - 5, step=3)
        def _(j):
            for k in range(3):
                buf = rings[k]
                gather_wait(j + k, buf)
                sc = scatter(j + k, buf)
                sc.wait()
                gather(j + k + 3, buf)

        for jj in range(CH - 5, CH):
            buf = rings[jj % 3]
            gather_wait(jj, buf)
            sc = scatter(jj, buf)
            sc.wait()
            if jj + 3 < CH:
                gather(jj + 3, buf)

    plsc.subcore_barrier()
    pltpu.sync_copy(acc.at[pl.ds(s * RPT, RPT)],
                    out_hbm.at[pl.ds(c * N + s * RPT, RPT)])

    @pl.when(s == NS - 1)
    def _():
        pltpu.sync_copy(acc.at[pl.ds(TAIL_OFF, TAIL)],
                        out_hbm.at[pl.ds(c * N + TAIL_OFF, TAIL)])


def _first_body(h32_ref, x_ref, w_ref, b_ref, root_ref,
                r2_ref, self_ref, dis_ref, inv_ref, eye):
    i = pl.program_id(0)

    @pl.when(i == 0)
    def _():
        r = lax.broadcasted_iota(jnp.int32, (BS, BS), 0)
        cc = lax.broadcasted_iota(jnp.int32, (BS, BS), 1)
        eye[...] = jnp.where(r == cc, 1.0, 0.0).astype(jnp.float32)

    deg_row = jnp.sum(h32_ref[0], axis=0, keepdims=True)[:, 0:BS] + 1.0
    deg = lax.dot_general(eye[...], deg_row, (((1,), (1,)), ((), ())),
                          preferred_element_type=jnp.float32)
    dis = lax.rsqrt(deg)
    inv = 1.0 / deg
    dis_ref[...] = dis
    inv_ref[...] = inv
    hl = jnp.dot(x_ref[...], w_ref[...],
                 preferred_element_type=jnp.float32) + b_ref[...]
    r2_ref[...] = dis * jnp.maximum(hl, 0.0)
    self_ref[...] = jnp.maximum(hl + root_ref[...], 0.0) * inv


def _tc_first(hist, x, Wl, bl, rootl):
    return pl.pallas_call(
        _first_body,
        grid=(NBLK,),
        in_specs=[
            pl.BlockSpec((1, NW, 1024), lambda i: (i, 0, 0)),
            pl.BlockSpec((BS, EMB), lambda i: (i, 0)),
            pl.BlockSpec((EMB, EMB), lambda i: (0, 0)),
            pl.BlockSpec((1, EMB), lambda i: (0, 0)),
            pl.BlockSpec((1, EMB), lambda i: (0, 0)),
        ],
        out_specs=[
            pl.BlockSpec((BS, EMB), lambda i: (i, 0)),
            pl.BlockSpec((BS, EMB), lambda i: (i, 0)),
            pl.BlockSpec((BS, 1), lambda i: (i, 0)),
            pl.BlockSpec((BS, 1), lambda i: (i, 0)),
        ],
        out_shape=[
            jax.ShapeDtypeStruct((N, EMB), jnp.float32),
            jax.ShapeDtypeStruct((N, EMB), jnp.float32),
            jax.ShapeDtypeStruct((N, 1), jnp.float32),
            jax.ShapeDtypeStruct((N, 1), jnp.float32),
        ],
        scratch_shapes=[pltpu.VMEM((BS, BS), jnp.float32)],
    )(hist, x, Wl, bl, rootl)


def _bn_apply(t, stats, scale_ref, bias_ref):
    mean = stats[0:1, :] * (1.0 / N)
    ex2 = stats[1:2, :] * (1.0 / N)
    var = ex2 - mean * mean
    rstd = lax.rsqrt(var + 1e-5)
    return (t - mean) * rstd * scale_ref[...] + bias_ref[...]


def _accum_phase(i, p0_ref, p1_ref, self_ref, dis_ref, t_buf, stats):
    t = dis_ref[...] * (p0_ref[...] + p1_ref[...]) + self_ref[...]
    t_buf[pl.ds(i * BS, BS), :] = t

    @pl.when(i == 0)
    def _():
        stats[...] = jnp.zeros_like(stats)

    stats[0:1, :] += jnp.sum(t, axis=0, keepdims=True)
    stats[1:2, :] += jnp.sum(t * t, axis=0, keepdims=True)


def _layer_body(p0_ref, p1_ref, self_ref, dis_ref, inv_ref, scale_ref,
                bias_ref, w_ref, b_ref, root_ref, r2_ref, self_out_ref,
                t_buf, stats):
    i = pl.program_id(0)

    @pl.when(i < NBLK)
    def _():
        _accum_phase(i, p0_ref, p1_ref, self_ref, dis_ref, t_buf, stats)

    @pl.when(i >= NBLK)
    def _():
        j = i - NBLK
        t = t_buf[pl.ds(j * BS, BS), :]
        h = jnp.maximum(_bn_apply(t, stats[...], scale_ref, bias_ref), 0.0)
        hl = jnp.dot(h, w_ref[...],
                     preferred_element_type=jnp.float32) + b_ref[...]
        r2_ref[...] = dis_ref[...] * jnp.maximum(hl, 0.0)
        self_out_ref[...] = jnp.maximum(hl + root_ref[...], 0.0) * inv_ref[...]


def _tc_layer(pflat, selfterm, dis, invdeg, scale_prev, bias_prev,
              Wl, bl, rootl):
    lo = lambda i: (jnp.minimum(i, NBLK - 1), 0)
    hi = lambda i: (NBLK + jnp.minimum(i, NBLK - 1), 0)
    ph2 = lambda i: (jnp.maximum(i - NBLK, 0), 0)
    both = lambda i: (jnp.where(i < NBLK, i, i - NBLK), 0)
    return pl.pallas_call(
        _layer_body,
        grid=(2 * NBLK,),
        in_specs=[
            pl.BlockSpec((BS, EMB), lo),
            pl.BlockSpec((BS, EMB), hi),
            pl.BlockSpec((BS, EMB), lo),
            pl.BlockSpec((BS, 1), both),
            pl.BlockSpec((BS, 1), ph2),
            pl.BlockSpec((1, EMB), lambda i: (0, 0)),
            pl.BlockSpec((1, EMB), lambda i: (0, 0)),
            pl.BlockSpec((EMB, EMB), lambda i: (0, 0)),
            pl.BlockSpec((1, EMB), lambda i: (0, 0)),
            pl.BlockSpec((1, EMB), lambda i: (0, 0)),
        ],
        out_specs=[
            pl.BlockSpec((BS, EMB), ph2),
            pl.BlockSpec((BS, EMB), ph2),
        ],
        out_shape=[
            jax.ShapeDtypeStruct((N, EMB), jnp.float32),
            jax.ShapeDtypeStruct((N, EMB), jnp.float32),
        ],
        scratch_shapes=[
            pltpu.VMEM((N, EMB), jnp.float32),
            pltpu.VMEM((2, EMB), jnp.float32),
        ],
    )(pflat, pflat, selfterm, dis, invdeg, scale_prev, bias_prev,
      Wl, bl, rootl)


def _tail_body(p0_ref, p1_ref, self_ref, dis_ref, scale_ref, bias_ref,
               batch_ref, wp_ref, bp_ref, out_ref, t_buf, stats,
               pooled, counts):
    i = pl.program_id(0)

    @pl.when(i < NBLK)
    def _():
        _accum_phase(i, p0_ref, p1_ref, self_ref, dis_ref, t_buf, stats)

    @pl.when(i >= NBLK)
    def _():
        j = i - NBLK
        t = t_buf[pl.ds(j * BS, BS), :]
        h = _bn_apply(t, stats[...], scale_ref, bias_ref)
        gids = lax.broadcasted_iota(jnp.int32, (NGRAPH, BS), 0)
        onehot = jnp.where(gids == batch_ref[0], 1.0, 0.0).astype(jnp.float32)

        @pl.when(j == 0)
        def _():
            pooled[...] = jnp.zeros_like(pooled)
            counts[...] = jnp.zeros_like(counts)

        pooled[...] += jnp.dot(onehot, h, preferred_element_type=jnp.float32)
        counts[...] += jnp.sum(onehot, axis=1, keepdims=True)

        @pl.when(j == NBLK - 1)
        def _():
            hg = pooled[...] / jnp.maximum(counts[...], 1.0)
            out_ref[...] = jnp.dot(
                hg, wp_ref[...],
                preferred_element_type=jnp.float32) + bp_ref[...]


def _tc_tail(pflat, selfterm, dis, scale4, bias4, batch_row, Wp, bp):
    lo = lambda i: (jnp.minimum(i, NBLK - 1), 0)
    hi = lambda i: (NBLK + jnp.minimum(i, NBLK - 1), 0)
    both = lambda i: (jnp.where(i < NBLK, i, i - NBLK), 0)
    return pl.pallas_call(
        _tail_body,
        grid=(2 * NBLK,),
        in_specs=[
            pl.BlockSpec((BS, EMB), lo),
            pl.BlockSpec((BS, EMB), hi),
            pl.BlockSpec((BS, EMB), lo),
            pl.BlockSpec((BS, 1), lo),
            pl.BlockSpec((1, EMB), lambda i: (0, 0)),
            pl.BlockSpec((1, EMB), lambda i: (0, 0)),
            pl.BlockSpec((1, 1, BS), lambda i: (jnp.maximum(i - NBLK, 0), 0, 0)),
            pl.BlockSpec((EMB, NCLASS), lambda i: (0, 0)),
            pl.BlockSpec((1, NCLASS), lambda i: (0, 0)),
        ],
        out_specs=pl.BlockSpec((NGRAPH, NCLASS), lambda i: (0, 0)),
        out_shape=jax.ShapeDtypeStruct((NGRAPH, NCLASS), jnp.float32),
        scratch_shapes=[
            pltpu.VMEM((N, EMB), jnp.float32),
            pltpu.VMEM((2, EMB), jnp.float32),
            pltpu.VMEM((NGRAPH, EMB), jnp.float32),
            pltpu.VMEM((NGRAPH, 1), jnp.float32),
        ],
    )(pflat, pflat, selfterm, dis, scale4, bias4, batch_row, Wp, bp)


def kernel(x, edge_index, batch, W, b, root, bn_scale, bn_bias, Wp, bp):
    src = edge_index[0]
    src3 = src.reshape(NW, NCH, CH, WIN)
    dst3 = edge_index[1].reshape(NW, NCH, CH, WIN)
    batch_row = batch.reshape(NBLK, 1, BS)
    zeros128 = jnp.zeros((N, EMB), jnp.float32)

    hist = _sc_degree(src)
    r2, selfterm, dis, invdeg = _tc_first(hist, x, W[0], b[0].reshape(1, EMB),
                                          root[0].reshape(1, EMB))
    for l in range(NLAYER - 1):
        pflat = _sc_aggregate(r2, src3, dst3, zeros128)
        r2, selfterm = _tc_layer(
            pflat, selfterm, dis, invdeg,
            bn_scale[l].reshape(1, EMB), bn_bias[l].reshape(1, EMB),
            W[l + 1], b[l + 1].reshape(1, EMB), root[l + 1].reshape(1, EMB))

    pflat = _sc_aggregate(r2, src3, dst3, zeros128)
    return _tc_tail(pflat, selfterm, dis,
                    bn_scale[NLAYER - 1].reshape(1, EMB),
                    bn_bias[NLAYER - 1].reshape(1, EMB),
                    batch_row, Wp, bp.reshape(1, NCLASS))


# speedup vs baseline: 17.6083x; 1.0223x over previous
import dataclasses
import functools

import jax
import jax.numpy as jnp
from jax import lax
from jax.experimental import pallas as pl
from jax.experimental.pallas import tpu as pltpu
from jax.experimental.pallas import tpu_sc as plsc

N = 10000
E = 320000
EMB = 128
NLAYER = 5
NGRAPH = 64
NCLASS = 128

NC = 2
NS = 16
NW = NC * NS
EW = E // NW
WIN = 100
NWIN = EW // WIN
CH = 20
NCH = NWIN // CH
RPT = 624
TAIL = N - NS * RPT
TAIL_OFF = NS * RPT

BS = 1000
NBLK = N // BS

_mesh = plsc.VectorSubcoreMesh(core_axis_name="c", subcore_axis_name="s",
                               num_cores=NC, num_subcores=NS)

_sc_params = pltpu.CompilerParams()
if "needs_layout_passes" in pltpu.CompilerParams.__dataclass_fields__:
    _sc_params = dataclasses.replace(_sc_params, needs_layout_passes=False)


@functools.partial(
    pl.kernel,
    out_type=jax.ShapeDtypeStruct((NBLK, NW, 1024), jnp.float32),
    mesh=_mesh,
    compiler_params=_sc_params,
    scratch_types=[
        pltpu.VMEM((EW,), jnp.int32),
        pltpu.VMEM((NBLK * 1024,), jnp.float32),
        pltpu.SemaphoreType.DMA,
    ],
)
def _sc_degree(src_hbm, out_hbm, sidx, hist, dsem):
    c = lax.axis_index("c")
    s = lax.axis_index("s")
    wid = c * NS + s
    pltpu.sync_copy(src_hbm.at[pl.ds(wid * EW, EW)], sidx)
    zeros16 = jnp.zeros((16,), jnp.float32)

    @pl.loop(0, NBLK * 1024, step=128)
    def _(i):
        for k in range(8):
            hist[pl.ds(i + 16 * k, 16)] = zeros16

    ones16 = jnp.ones((16,), jnp.float32)
    bs16 = jnp.full((16,), BS, jnp.int32)

    @pl.loop(0, EW, step=80)
    def _(i):
        for k in range(5):
            idx = sidx[pl.ds(i + 16 * k, 16)]
            q = lax.div(idx, bs16)
            pos = idx + q * (1024 - BS)
            plsc.addupdate_scatter(hist, [pos], ones16)

    for ib in range(NBLK):
        pltpu.async_copy(hist.at[pl.ds(ib * 1024, 1024)],
                         out_hbm.at[ib].at[wid], dsem)
    for ib in range(NBLK):
        pltpu.make_async_copy(hist.at[pl.ds(ib * 1024, 1024)],
                              out_hbm.at[ib].at[wid], dsem).wait()


@functools.partial(
    pl.kernel,
    out_type=jax.ShapeDtypeStruct((NC * N, EMB), jnp.float32),
    mesh=_mesh,
    scratch_types=[
        pltpu.VMEM((CH, WIN), jnp.int32),
        pltpu.VMEM((CH, WIN), jnp.int32),
        pltpu.VMEM((WIN, EMB), jnp.float32),
        pltpu.VMEM((WIN, EMB), jnp.float32),
        pltpu.VMEM((WIN, EMB), jnp.float32),
        pltpu.VMEM_SHARED((N, EMB), jnp.float32),
        pltpu.SemaphoreType.DMA,
        pltpu.SemaphoreType.DMA,
    ],
)
def _sc_aggregate(r2_hbm, src3_hbm, dst3_hbm, zeros_hbm, out_hbm,
                  sidx2, didx2, rows_a, rows_b, rows_c, acc, gsem, ssem):
    c = lax.axis_index("c")
    s = lax.axis_index("s")
    pltpu.sync_copy(zeros_hbm.at[pl.ds(s * RPT, RPT)], acc.at[pl.ds(s * RPT, RPT)])

    @pl.when(s == NS - 1)
    def _():
        pltpu.sync_copy(zeros_hbm.at[pl.ds(TAIL_OFF, TAIL)],
                        acc.at[pl.ds(TAIL_OFF, TAIL)])

    wid = c * NS + s
    plsc.subcore_barrier()

    def gather(j, buf):
        return pltpu.async_copy(r2_hbm.at[sidx2.at[j]], buf, gsem)

    def gather_wait(j, buf):
        pltpu.make_async_copy(r2_hbm.at[sidx2.at[j]], buf, gsem).wait()

    def scatter(j, buf):
        return pltpu.async_copy(buf, acc.at[didx2.at[j]], ssem, add=True)

    rings = (rows_a, rows_b, rows_c)

    @pl.loop(0, NCH)
    def _(ch):
        si = pltpu.async_copy(src3_hbm.at[wid].at[ch], sidx2, gsem)
        di = pltpu.async_copy(dst3_hbm.at[wid].at[ch], didx2, ssem)
        si.wait()
        di.wait()
        gather(0, rows_a)
        gather(1, rows_b)
        gather(2, rows_c)

        @pl.loop(0, CH - 5, step=3)
        def _(j):
            for k in range(3):
                buf = rings[k]
                gather_wait(j + k, buf)
                sc = scatter(j + k, buf)
                sc.wait()
                gather(j + k + 3, buf)

        for jj in range(CH - 5, CH):
            buf = rings[jj % 3]
            gather_wait(jj, buf)
            sc = scatter(jj, buf)
            sc.wait()
            if jj + 3 < CH:
                gather(jj + 3, buf)

    plsc.subcore_barrier()
    pltpu.sync_copy(acc.at[pl.ds(s * RPT, RPT)],
                    out_hbm.at[pl.ds(c * N + s * RPT, RPT)])

    @pl.when(s == NS - 1)
    def _():
        pltpu.sync_copy(acc.at[pl.ds(TAIL_OFF, TAIL)],
                        out_hbm.at[pl.ds(c * N + TAIL_OFF, TAIL)])


def _first_body(h32_ref, x_ref, w_ref, b_ref, root_ref,
                r2_ref, self_ref, dis_ref, inv_ref, eye):
    i = pl.program_id(0)

    @pl.when(i == 0)
    def _():
        r = lax.broadcasted_iota(jnp.int32, (BS, BS), 0)
        cc = lax.broadcasted_iota(jnp.int32, (BS, BS), 1)
        eye[...] = jnp.where(r == cc, 1.0, 0.0).astype(jnp.float32)

    deg_row = jnp.sum(h32_ref[0], axis=0, keepdims=True)[:, 0:BS] + 1.0
    deg = lax.dot_general(eye[...], deg_row, (((1,), (1,)), ((), ())),
                          preferred_element_type=jnp.float32)
    dis = lax.rsqrt(deg)
    inv = 1.0 / deg
    dis_ref[...] = dis
    inv_ref[...] = inv
    hl = jnp.dot(x_ref[...], w_ref[...],
                 preferred_element_type=jnp.float32) + b_ref[...]
    r2_ref[...] = dis * jnp.maximum(hl, 0.0)
    self_ref[...] = jnp.maximum(hl + root_ref[...], 0.0) * inv


def _tc_first(hist, x, Wl, bl, rootl):
    return pl.pallas_call(
        _first_body,
        grid=(NBLK,),
        in_specs=[
            pl.BlockSpec((1, NW, 1024), lambda i: (i, 0, 0)),
            pl.BlockSpec((BS, EMB), lambda i: (i, 0)),
            pl.BlockSpec((EMB, EMB), lambda i: (0, 0)),
            pl.BlockSpec((1, EMB), lambda i: (0, 0)),
            pl.BlockSpec((1, EMB), lambda i: (0, 0)),
        ],
        out_specs=[
            pl.BlockSpec((BS, EMB), lambda i: (i, 0)),
            pl.BlockSpec((BS, EMB), lambda i: (i, 0)),
            pl.BlockSpec((BS, 1), lambda i: (i, 0)),
            pl.BlockSpec((BS, 1), lambda i: (i, 0)),
        ],
        out_shape=[
            jax.ShapeDtypeStruct((N, EMB), jnp.float32),
            jax.ShapeDtypeStruct((N, EMB), jnp.float32),
            jax.ShapeDtypeStruct((N, 1), jnp.float32),
            jax.ShapeDtypeStruct((N, 1), jnp.float32),
        ],
        scratch_shapes=[pltpu.VMEM((BS, BS), jnp.float32)],
    )(hist, x, Wl, bl, rootl)


def _bn_apply(t, stats, scale_ref, bias_ref):
    mean = stats[0:1, :] * (1.0 / N)
    ex2 = stats[1:2, :] * (1.0 / N)
    var = ex2 - mean * mean
    rstd = lax.rsqrt(var + 1e-5)
    return (t - mean) * rstd * scale_ref[...] + bias_ref[...]


def _accum_phase(i, p0_ref, p1_ref, self_ref, dis_ref, t_buf, stats):
    t = dis_ref[...] * (p0_ref[...] + p1_ref[...]) + self_ref[...]
    t_buf[pl.ds(i * BS, BS), :] = t

    @pl.when(i == 0)
    def _():
        stats[...] = jnp.zeros_like(stats)

    stats[0:1, :] += jnp.sum(t, axis=0, keepdims=True)
    stats[1:2, :] += jnp.sum(t * t, axis=0, keepdims=True)


def _layer_body(p0_ref, p1_ref, self_ref, dis_ref, inv_ref, scale_ref,
                bias_ref, w_ref, b_ref, root_ref, r2_ref, self_out_ref,
                t_buf, stats):
    i = pl.program_id(0)

    @pl.when(i < NBLK)
    def _():
        _accum_phase(i, p0_ref, p1_ref, self_ref, dis_ref, t_buf, stats)

    @pl.when(i >= NBLK)
    def _():
        j = i - NBLK
        t = t_buf[pl.ds(j * BS, BS), :]
        h = jnp.maximum(_bn_apply(t, stats[...], scale_ref, bias_ref), 0.0)
        hl = jnp.dot(h, w_ref[...],
                     preferred_element_type=jnp.float32) + b_ref[...]
        r2_ref[...] = dis_ref[...] * jnp.maximum(hl, 0.0)
        self_out_ref[...] = jnp.maximum(hl + root_ref[...], 0.0) * inv_ref[...]


def _tc_layer(pflat, selfterm, dis, invdeg, scale_prev, bias_prev,
              Wl, bl, rootl):
    lo = lambda i: (jnp.minimum(i, NBLK - 1), 0)
    hi = lambda i: (NBLK + jnp.minimum(i, NBLK - 1), 0)
    ph2 = lambda i: (jnp.maximum(i - NBLK, 0), 0)
    both = lambda i: (jnp.where(i < NBLK, i, i - NBLK), 0)
    return pl.pallas_call(
        _layer_body,
        grid=(2 * NBLK,),
        in_specs=[
            pl.BlockSpec((BS, EMB), lo),
            pl.BlockSpec((BS, EMB), hi),
            pl.BlockSpec((BS, EMB), lo),
            pl.BlockSpec((BS, 1), both),
            pl.BlockSpec((BS, 1), ph2),
            pl.BlockSpec((1, EMB), lambda i: (0, 0)),
            pl.BlockSpec((1, EMB), lambda i: (0, 0)),
            pl.BlockSpec((EMB, EMB), lambda i: (0, 0)),
            pl.BlockSpec((1, EMB), lambda i: (0, 0)),
            pl.BlockSpec((1, EMB), lambda i: (0, 0)),
        ],
        out_specs=[
            pl.BlockSpec((BS, EMB), ph2),
            pl.BlockSpec((BS, EMB), ph2),
        ],
        out_shape=[
            jax.ShapeDtypeStruct((N, EMB), jnp.float32),
            jax.ShapeDtypeStruct((N, EMB), jnp.float32),
        ],
        scratch_shapes=[
            pltpu.VMEM((N, EMB), jnp.float32),
            pltpu.VMEM((2, EMB), jnp.float32),
        ],
    )(pflat, pflat, selfterm, dis, invdeg, scale_prev, bias_prev,
      Wl, bl, rootl)


def _tail_body(p0_ref, p1_ref, self_ref, dis_ref, scale_ref, bias_ref,
               batch_ref, wp_ref, bp_ref, out_ref, t_buf, stats,
               pooled, counts):
    i = pl.program_id(0)

    @pl.when(i < NBLK)
    def _():
        _accum_phase(i, p0_ref, p1_ref, self_ref, dis_ref, t_buf, stats)

    @pl.when(i >= NBLK)
    def _():
        j = i - NBLK
        t = t_buf[pl.ds(j * BS, BS), :]
        h = _bn_apply(t, stats[...], scale_ref, bias_ref)
        gids = lax.broadcasted_iota(jnp.int32, (NGRAPH, BS), 0)
        onehot = jnp.where(gids == batch_ref[0], 1.0, 0.0).astype(jnp.float32)

        @pl.when(j == 0)
        def _():
            pooled[...] = jnp.zeros_like(pooled)
            counts[...] = jnp.zeros_like(counts)

        pooled[...] += jnp.dot(onehot, h, preferred_element_type=jnp.float32)
        counts[...] += jnp.sum(onehot, axis=1, keepdims=True)

        @pl.when(j == NBLK - 1)
        def _():
            hg = pooled[...] / jnp.maximum(counts[...], 1.0)
            out_ref[...] = jnp.dot(
                hg, wp_ref[...],
                preferred_element_type=jnp.float32) + bp_ref[...]


def _tc_tail(pflat, selfterm, dis, scale4, bias4, batch_row, Wp, bp):
    lo = lambda i: (jnp.minimum(i, NBLK - 1), 0)
    hi = lambda i: (NBLK + jnp.minimum(i, NBLK - 1), 0)
    both = lambda i: (jnp.where(i < NBLK, i, i - NBLK), 0)
    return pl.pallas_call(
        _tail_body,
        grid=(2 * NBLK,),
        in_specs=[
            pl.BlockSpec((BS, EMB), lo),
            pl.BlockSpec((BS, EMB), hi),
            pl.BlockSpec((BS, EMB), lo),
            pl.BlockSpec((BS, 1), lo),
            pl.BlockSpec((1, EMB), lambda i: (0, 0)),
            pl.BlockSpec((1, EMB), lambda i: (0, 0)),
            pl.BlockSpec((1, 1, BS), lambda i: (jnp.maximum(i - NBLK, 0), 0, 0)),
            pl.BlockSpec((EMB, NCLASS), lambda i: (0, 0)),
            pl.BlockSpec((1, NCLASS), lambda i: (0, 0)),
        ],
        out_specs=pl.BlockSpec((NGRAPH, NCLASS), lambda i: (0, 0)),
        out_shape=jax.ShapeDtypeStruct((NGRAPH, NCLASS), jnp.float32),
        scratch_shapes=[
            pltpu.VMEM((N, EMB), jnp.float32),
            pltpu.VMEM((2, EMB), jnp.float32),
            pltpu.VMEM((NGRAPH, EMB), jnp.float32),
            pltpu.VMEM((NGRAPH, 1), jnp.float32),
        ],
    )(pflat, pflat, selfterm, dis, scale4, bias4, batch_row, Wp, bp)


def kernel(x, edge_index, batch, W, b, root, bn_scale, bn_bias, Wp, bp):
    src = edge_index[0]
    src3 = src.reshape(NW, NCH, CH, WIN)
    dst3 = edge_index[1].reshape(NW, NCH, CH, WIN)
    batch_row = batch.reshape(NBLK, 1, BS)
    zeros128 = jnp.zeros((N, EMB), jnp.float32)

    hist = _sc_degree(src)
    r2, selfterm, dis, invdeg = _tc_first(hist, x, W[0], b[0].reshape(1, EMB),
                                          root[0].reshape(1, EMB))
    for l in range(NLAYER - 1):
        pflat = _sc_aggregate(r2, src3, dst3, zeros128)
        r2, selfterm = _tc_layer(
            pflat, selfterm, dis, invdeg,
            bn_scale[l].reshape(1, EMB), bn_bias[l].reshape(1, EMB),
            W[l + 1], b[l + 1].reshape(1, EMB), root[l + 1].reshape(1, EMB))

    pflat = _sc_aggregate(r2, src3, dst3, zeros128)
    return _tc_tail(pflat, selfterm, dis,
                    bn_scale[NLAYER - 1].reshape(1, EMB),
                    bn_bias[NLAYER - 1].reshape(1, EMB),
                    batch_row, Wp, bp.reshape(1, NCLASS))
